# baseline (speedup 1.0000x reference)
import functools

import jax
import jax.numpy as jnp
from jax import lax
from jax.experimental import pallas as pl
from jax.experimental.pallas import tpu as pltpu
from jax.experimental.pallas import tpu_sc as plsc

_EPS = 1e-5
_K = 16
_BIG = 2**30


def _knn_body(tr_ref, ptsT_ref, b2d_ref, bcol_ref, idx_ref, d2_ref, *, rb, ct, n):
    i = pl.program_id(0)
    tlo = tr_ref[2 * i]
    thi = tr_ref[2 * i + 1]
    prow = ptsT_ref[:, pl.ds(i * rb, rb)]
    sqrow = lax.dot_general(
        prow * prow, jnp.ones((8, 1), jnp.float32),
        (((0,), (0,)), ((), ())), preferred_element_type=jnp.float32)
    bcol = bcol_ref[...]
    m0 = jnp.full((rb, 1), jnp.inf, jnp.float32)
    a0 = jnp.full((rb, 1), _BIG, jnp.int32)
    iota_c = lax.broadcasted_iota(jnp.int32, (rb, ct), 1)
    lane_k = lax.broadcasted_iota(jnp.int32, (rb, _K), 1)

    def merge(carry, mt, cc):
        m, am = carry
        am2 = jnp.where(mt < m, cc,
                        jnp.where(mt == m, jnp.minimum(am, cc), am))
        return jnp.minimum(m, mt), am2

    def init_body(t, carry):
        off = pl.multiple_of(t * ct, ct)
        ptile = ptsT_ref[:, pl.ds(off, ct)]
        pp = lax.dot_general(prow, ptile, (((0,), (0,)), ((), ())),
                             preferred_element_type=jnp.float32)
        sqcol = jnp.sum(ptile * ptile, axis=0, keepdims=True)
        d2 = sqrow + sqcol - 2.0 * pp
        same = bcol == b2d_ref[:, pl.ds(off, ct)]
        tile = jnp.where(same, d2, jnp.inf)
        d2_ref[:, pl.ds(off, ct)] = tile
        it = iota_c + t * ct
        mt = jnp.min(tile, axis=1, keepdims=True)
        cc = jnp.min(jnp.where(tile == mt, it, _BIG), axis=1, keepdims=True)
        return merge(carry, mt, cc)

    m, am = lax.fori_loop(tlo, thi, init_body, (m0, a0))
    acc = jnp.where(lane_k == 0, am, 0)

    def kstep(k, carry):
        acc, vprev, aprev = carry

        def sbody(t, c):
            off = pl.multiple_of(t * ct, ct)
            tile = d2_ref[:, pl.ds(off, ct)]
            it = iota_c + t * ct
            valid = (tile > vprev) | ((tile == vprev) & (it > aprev))
            mt = jnp.min(jnp.where(valid, tile, jnp.inf), axis=1,
                         keepdims=True)
            cc = jnp.min(jnp.where(valid & (tile == mt), it, _BIG), axis=1,
                         keepdims=True)
            return merge(c, mt, cc)

        m, am = lax.fori_loop(tlo, thi, sbody, (m0, a0))
        acc = jnp.where(lane_k == k, am, acc)
        return acc, m, am

    acc, _, _ = lax.fori_loop(1, _K, kstep, (acc, m, am))
    idx_ref[...] = acc


def _knn(ptsT, b2d, bcol, tr, n):
    rb = 256 if n % 256 == 0 else n
    ct = 1024 if n % 1024 == 0 else n
    grid_spec = pltpu.PrefetchScalarGridSpec(
        num_scalar_prefetch=1,
        grid=(n // rb,),
        in_specs=[
            pl.BlockSpec((8, n), lambda i, *_: (0, 0)),
            pl.BlockSpec((1, n), lambda i, *_: (0, 0)),
            pl.BlockSpec((rb, 1), lambda i, *_: (i, 0)),
        ],
        out_specs=pl.BlockSpec((rb, _K), lambda i, *_: (i, 0)),
        scratch_shapes=[pltpu.VMEM((rb, n), jnp.float32)],
    )
    return pl.pallas_call(
        functools.partial(_knn_body, rb=rb, ct=ct, n=n),
        grid_spec=grid_spec,
        out_shape=jax.ShapeDtypeStruct((n, _K), jnp.int32),
    )(tr, ptsT, b2d, bcol)


def _tile_ranges(b, n, rb, ct):
    nblk = n // rb
    ntiles = n // ct
    bb = b.reshape(nblk, rb)
    bcast = b[None, :]
    lo_col = jnp.sum((bcast < bb[:, 0][:, None]).astype(jnp.int32), axis=1)
    hi_col = jnp.sum((bcast <= bb[:, -1][:, None]).astype(jnp.int32), axis=1)
    vals = jnp.arange(8, dtype=jnp.int32)
    hist = jnp.sum((bcast == vals[:, None]).astype(jnp.int32), axis=1)
    minsz = jnp.min(jnp.where(hist > 0, hist, n))
    tlo = jnp.where(minsz < _K, 0, lo_col // ct)
    thi = jnp.where(minsz < _K, ntiles, (hi_col + ct - 1) // ct)
    return jnp.stack([tlo, thi], axis=1).reshape(-1).astype(jnp.int32)


def _gmat_body(x_ref, pts_ref, feat_ref, cwT_ref, w3T_ref, l1bT_ref,
               g_ref, p2_ref, f3_ref):
    g_ref[...] = jnp.dot(x_ref[...], cwT_ref[...],
                         preferred_element_type=jnp.float32)
    p2_ref[...] = jnp.dot(pts_ref[...], w3T_ref[...],
                          preferred_element_type=jnp.float32)
    f3_ref[...] = jnp.dot(feat_ref[...], l1bT_ref[...],
                          preferred_element_type=jnp.float32)


def _gmat(x_cat, pts_pad, feature, cwT, w3T, l1bT, n, cin, inner):
    rbg = 512 if n % 512 == 0 else n
    c3 = cin + 3
    return pl.pallas_call(
        _gmat_body,
        grid=(n // rbg,),
        in_specs=[
            pl.BlockSpec((rbg, c3), lambda i: (i, 0)),
            pl.BlockSpec((rbg, 8), lambda i: (i, 0)),
            pl.BlockSpec((rbg, cin), lambda i: (i, 0)),
            pl.BlockSpec((c3, inner), lambda i: (0, 0)),
            pl.BlockSpec((8, inner), lambda i: (0, 0)),
            pl.BlockSpec((cin, inner), lambda i: (0, 0)),
        ],
        out_specs=[
            pl.BlockSpec((rbg, inner), lambda i: (i, 0)),
            pl.BlockSpec((rbg, inner), lambda i: (i, 0)),
            pl.BlockSpec((rbg, inner), lambda i: (i, 0)),
        ],
        out_shape=[
            jax.ShapeDtypeStruct((n, inner), jnp.float32),
            jax.ShapeDtypeStruct((n, inner), jnp.float32),
            jax.ShapeDtypeStruct((n, inner), jnp.float32),
        ],
    )(x_cat, pts_pad, feature, cwT, w3T, l1bT)


def _gather_rows(table, idx_flat):
    nrows = idx_flat.shape[0]
    d = table.shape[1]
    nw = 32
    per_w = nrows // nw
    ch = 128
    nch = per_w // ch
    mesh = plsc.VectorSubcoreMesh(core_axis_name="c", subcore_axis_name="s")

    @functools.partial(
        pl.kernel, mesh=mesh,
        out_type=jax.ShapeDtypeStruct((nrows, d), jnp.float32),
        scratch_types=[
            pltpu.VMEM((ch,), jnp.int32),
            pltpu.VMEM((ch, d), jnp.float32),
            pltpu.VMEM((ch,), jnp.int32),
            pltpu.VMEM((ch, d), jnp.float32),
            pltpu.SemaphoreType.DMA,
            pltpu.SemaphoreType.DMA,
        ],
    )
    def gk(table_hbm, idx_hbm, out_hbm, idx0, rows0, idx1, rows1, sem0, sem1):
        wid = lax.axis_index("s") * 2 + lax.axis_index("c")
        base = wid * per_w
        idx_v = [idx0, idx1]
        rows_v = [rows0, rows1]
        sems = [sem0, sem1]

        def start(c, slot):
            off = base + c * ch
            pltpu.sync_copy(idx_hbm.at[pl.ds(off, ch)], idx_v[slot])
            pltpu.async_copy(table_hbm.at[idx_v[slot]], rows_v[slot], sems[slot])

        def drain(c, slot):
            off = base + c * ch
            pltpu.make_async_copy(table_hbm.at[idx_v[slot]], rows_v[slot],
                                  sems[slot]).wait()
            pltpu.sync_copy(rows_v[slot], out_hbm.at[pl.ds(off, ch)])

        start(0, 0)
        def body(c, carry):
            slot = lax.rem(c, 2)
            nslot = 1 - slot
            @pl.when(c + 1 < nch)
            def _():
                jax.lax.switch(nslot, [lambda: start(c + 1, 0),
                                       lambda: start(c + 1, 1)])
            jax.lax.switch(slot, [lambda: drain(c, 0), lambda: drain(c, 1)])
            return carry

        lax.fori_loop(0, nch, body, 0)

    return gk(table, idx_flat)


def _stats1_body(h_ref, p2_ref, out_ref, *, inner):
    i = pl.program_id(0)
    hs = None
    h2s = None
    for k in range(_K):
        hk = h_ref[:, k, :]
        hs = hk if hs is None else hs + hk
        h2s = hk * hk if h2s is None else h2s + hk * hk
    p2 = p2_ref[...]
    s1 = jnp.sum(hs, axis=0, keepdims=True)
    s2 = jnp.sum(h2s, axis=0, keepdims=True)
    s3 = jnp.sum(p2 * hs, axis=0, keepdims=True)
    s4 = jnp.sum(p2, axis=0, keepdims=True)
    s5 = jnp.sum(p2 * p2, axis=0, keepdims=True)
    contrib = jnp.concatenate(
        [s1, s2, s3, s4, s5, jnp.zeros((3, inner), jnp.float32)], axis=0)

    @pl.when(i == 0)
    def _():
        out_ref[...] = jnp.zeros_like(out_ref)

    out_ref[...] += contrib


def _stats1(h3, p2, n, inner):
    pb = 128 if n % 128 == 0 else n
    return pl.pallas_call(
        functools.partial(_stats1_body, inner=inner),
        grid=(n // pb,),
        in_specs=[
            pl.BlockSpec((pb, _K, inner), lambda i: (i, 0, 0)),
            pl.BlockSpec((pb, inner), lambda i: (i, 0)),
        ],
        out_specs=pl.BlockSpec((8, inner), lambda i: (0, 0)),
        out_shape=jax.ShapeDtypeStruct((8, inner), jnp.float32),
    )(h3, p2)


def _passb_body(h_ref, p2_ref, a_ref, c_ref, y_ref, w_ref, *, pb, inner):
    a = a_ref[...]
    u = c_ref[...] - a * p2_ref[...]
    s = a * h_ref[:, 0, :] + u
    y = None
    wcols = []
    for k in range(_K):
        xk = a * h_ref[:, k, :] + u
        wk = jnp.sum(xk * s, axis=1, keepdims=True)
        wcols.append(wk)
        yk = xk * wk
        y = yk if y is None else y + yk
    w_ref[...] = jnp.concatenate(wcols, axis=1)
    y_ref[...] = y


def _passb(h3, p2, a1, c1, n, inner):
    pb = 128 if n % 128 == 0 else n
    return pl.pallas_call(
        functools.partial(_passb_body, pb=pb, inner=inner),
        grid=(n // pb,),
        in_specs=[
            pl.BlockSpec((pb, _K, inner), lambda i: (i, 0, 0)),
            pl.BlockSpec((pb, inner), lambda i: (i, 0)),
            pl.BlockSpec((1, inner), lambda i: (0, 0)),
            pl.BlockSpec((1, inner), lambda i: (0, 0)),
        ],
        out_specs=[
            pl.BlockSpec((pb, inner), lambda i: (i, 0)),
            pl.BlockSpec((pb, _K), lambda i: (i, 0)),
        ],
        out_shape=[
            jax.ShapeDtypeStruct((n, inner), jnp.float32),
            jax.ShapeDtypeStruct((n, _K), jnp.float32),
        ],
    )(h3, p2, a1, c1)


def _passc_body(h2_ref, w_ref, z_ref, acc_ref, *, inner):
    i = pl.program_id(0)
    z = None
    for k in range(_K):
        zk = h2_ref[:, k, :] * w_ref[:, k:k + 1]
        z = zk if z is None else z + zk
    z_ref[...] = z
    contrib = jnp.concatenate(
        [jnp.sum(z, axis=0, keepdims=True),
         jnp.sum(z * z, axis=0, keepdims=True),
         jnp.zeros((6, inner), jnp.float32)], axis=0)

    @pl.when(i == 0)
    def _():
        acc_ref[...] = jnp.zeros_like(acc_ref)

    acc_ref[...] += contrib


def _passc(h23, w, n, inner):
    pb = 128 if n % 128 == 0 else n
    return pl.pallas_call(
        functools.partial(_passc_body, inner=inner),
        grid=(n // pb,),
        in_specs=[
            pl.BlockSpec((pb, _K, inner), lambda i: (i, 0, 0)),
            pl.BlockSpec((pb, _K), lambda i: (i, 0)),
        ],
        out_specs=[
            pl.BlockSpec((pb, inner), lambda i: (i, 0)),
            pl.BlockSpec((8, inner), lambda i: (0, 0)),
        ],
        out_shape=[
            jax.ShapeDtypeStruct((n, inner), jnp.float32),
            jax.ShapeDtypeStruct((8, inner), jnp.float32),
        ],
    )(h23, w)


def _passd1_body(z_ref, f3_ref, a2_ref, c2_ref, l1aT_ref, b1_ref,
                 t_ref, acc_ref, *, cin):
    i = pl.program_id(0)
    r = jnp.maximum(a2_ref[...] * z_ref[...] + c2_ref[...], 0.0)
    t = (jnp.dot(r, l1aT_ref[...], preferred_element_type=jnp.float32)
         + f3_ref[...] + b1_ref[...])
    t_ref[...] = t
    contrib = jnp.concatenate(
        [jnp.sum(t, axis=0, keepdims=True),
         jnp.sum(t * t, axis=0, keepdims=True),
         jnp.zeros((6, cin), jnp.float32)], axis=0)

    @pl.when(i == 0)
    def _():
        acc_ref[...] = jnp.zeros_like(acc_ref)

    acc_ref[...] += contrib


def _passd1(z, f3, a2, c2, l1aT, b1, n, cin, inner):
    rbg = 512 if n % 512 == 0 else n
    return pl.pallas_call(
        functools.partial(_passd1_body, cin=cin),
        grid=(n // rbg,),
        in_specs=[
            pl.BlockSpec((rbg, inner), lambda i: (i, 0)),
            pl.BlockSpec((rbg, cin), lambda i: (i, 0)),
            pl.BlockSpec((1, inner), lambda i: (0, 0)),
            pl.BlockSpec((1, inner), lambda i: (0, 0)),
            pl.BlockSpec((inner, cin), lambda i: (0, 0)),
            pl.BlockSpec((1, cin), lambda i: (0, 0)),
        ],
        out_specs=[
            pl.BlockSpec((rbg, cin), lambda i: (i, 0)),
            pl.BlockSpec((8, cin), lambda i: (0, 0)),
        ],
        out_shape=[
            jax.ShapeDtypeStruct((n, cin), jnp.float32),
            jax.ShapeDtypeStruct((8, cin), jnp.float32),
        ],
    )(z, f3, a2, c2, l1aT, b1)


def _passd2_body(t_ref, a3_ref, c3_ref, l2T_ref, b2_ref, o_ref):
    r = jnp.maximum(a3_ref[...] * t_ref[...] + c3_ref[...], 0.0)
    o_ref[...] = (jnp.dot(r, l2T_ref[...], preferred_element_type=jnp.float32)
                  + b2_ref[...])


def _passd2(t, a3, c3, l2T, b2, n, cin):
    rbg = 512 if n % 512 == 0 else n
    return pl.pallas_call(
        _passd2_body,
        grid=(n // rbg,),
        in_specs=[
            pl.BlockSpec((rbg, cin), lambda i: (i, 0)),
            pl.BlockSpec((1, cin), lambda i: (0, 0)),
            pl.BlockSpec((1, cin), lambda i: (0, 0)),
            pl.BlockSpec((cin, cin), lambda i: (0, 0)),
            pl.BlockSpec((1, cin), lambda i: (0, 0)),
        ],
        out_specs=pl.BlockSpec((rbg, cin), lambda i: (i, 0)),
        out_shape=jax.ShapeDtypeStruct((n, cin), jnp.float32),
    )(t, a3, c3, l2T, b2)


def kernel(coords, points, feature, conv_w, bn1_g, bn1_b, bn2_g, bn2_b,
           lin1_w, lin1_b, bn3_g, bn3_b, lin2_w, lin2_b):
    n, cin = feature.shape
    inner = conv_w.shape[0]
    f32 = jnp.float32
    b = coords[:, 3].astype(jnp.int32)

    ptsT = jnp.concatenate([points.T, jnp.zeros((5, n), f32)], axis=0)
    rb = 256 if n % 256 == 0 else n
    ct = 1024 if n % 1024 == 0 else n
    tr = _tile_ranges(b, n, rb, ct)
    idx = _knn(ptsT, b[None, :], b[:, None], tr, n)

    x_cat = jnp.concatenate([feature, points], axis=1)
    pts_pad = jnp.concatenate([points, jnp.zeros((n, 5), f32)], axis=1)
    w3T = jnp.concatenate(
        [conv_w[:, cin:].T, jnp.zeros((5, inner), f32)], axis=0)
    g, p2, f3 = _gmat(x_cat, pts_pad, feature, conv_w.T, w3T,
                      lin1_w[:, inner:].T, n, cin, inner)

    idx_flat = idx.reshape(-1)
    h3 = _gather_rows(g, idx_flat).reshape(n, _K, inner)
    s = _stats1(h3, p2, n, inner)
    nk = jnp.float32(n * _K)
    mean1 = (s[0] - _K * s[3]) / nk
    ex2 = (s[1] - 2.0 * s[2] + _K * s[4]) / nk
    var1 = ex2 - mean1 * mean1
    a1 = bn1_g / jnp.sqrt(var1 + _EPS)
    c1 = bn1_b - a1 * mean1

    y, w = _passb(h3, p2, a1[None], c1[None], n, inner)

    h23 = _gather_rows(y, idx_flat).reshape(n, _K, inner)
    z, acc2 = _passc(h23, w, n, inner)
    mean2 = acc2[0] / n
    var2 = acc2[1] / n - mean2 * mean2
    a2 = bn2_g / jnp.sqrt(var2 + _EPS)
    c2 = bn2_b - a2 * mean2

    t, acc3 = _passd1(z, f3, a2[None], c2[None], lin1_w[:, :inner].T,
                      lin1_b[None], n, cin, inner)
    mean3 = acc3[0] / n
    var3 = acc3[1] / n - mean3 * mean3
    a3 = bn3_g / jnp.sqrt(var3 + _EPS)
    c3 = bn3_b - a3 * mean3

    return _passd2(t, a3[None], c3[None], lin2_w.T, lin2_b[None], n, cin)


# speedup vs baseline: 1.2784x; 1.2784x over previous
import functools

import jax
import jax.numpy as jnp
from jax import lax
from jax.experimental import pallas as pl
from jax.experimental.pallas import tpu as pltpu
from jax.experimental.pallas import tpu_sc as plsc

_EPS = 1e-5
_K = 16
_BIG = 2**30


def _knn_body(tr_ref, ptsT_ref, b2d_ref, bcol_ref, idx_ref, d2_ref, *, rb, ct, n):
    i = pl.program_id(0)
    tlo = tr_ref[2 * i]
    thi = tr_ref[2 * i + 1]
    prow = ptsT_ref[:, pl.ds(i * rb, rb)]
    sqrow = lax.dot_general(
        prow * prow, jnp.ones((8, 1), jnp.float32),
        (((0,), (0,)), ((), ())), preferred_element_type=jnp.float32)
    bcol = bcol_ref[...]
    m0 = jnp.full((rb, 1), jnp.inf, jnp.float32)
    a0 = jnp.full((rb, 1), _BIG, jnp.int32)
    iota_c = lax.broadcasted_iota(jnp.int32, (rb, ct), 1)
    lane_k = lax.broadcasted_iota(jnp.int32, (rb, _K), 1)

    def merge(carry, mt, cc):
        m, am = carry
        am2 = jnp.where(mt < m, cc,
                        jnp.where(mt == m, jnp.minimum(am, cc), am))
        return jnp.minimum(m, mt), am2

    def init_body(t, carry):
        off = pl.multiple_of(t * ct, ct)
        ptile = ptsT_ref[:, pl.ds(off, ct)]
        pp = lax.dot_general(prow, ptile, (((0,), (0,)), ((), ())),
                             preferred_element_type=jnp.float32)
        sqcol = jnp.sum(ptile * ptile, axis=0, keepdims=True)
        d2 = sqrow + sqcol - 2.0 * pp
        same = bcol == b2d_ref[:, pl.ds(off, ct)]
        tile = jnp.where(same, d2, jnp.inf)
        d2_ref[:, pl.ds(off, ct)] = tile
        it = iota_c + t * ct
        mt = jnp.min(tile, axis=1, keepdims=True)
        cc = jnp.min(jnp.where(tile == mt, it, _BIG), axis=1, keepdims=True)
        return merge(carry, mt, cc)

    m, am = lax.fori_loop(tlo, thi, init_body, (m0, a0))
    acc = jnp.where(lane_k == 0, am, 0)

    def kstep(k, carry):
        acc, aprev = carry

        def sbody(t, c):
            off = pl.multiple_of(t * ct, ct)
            tile = d2_ref[:, pl.ds(off, ct)]
            it = iota_c + t * ct
            tile = jnp.where(it == aprev, jnp.inf, tile)
            d2_ref[:, pl.ds(off, ct)] = tile
            mt = jnp.min(tile, axis=1, keepdims=True)
            cc = jnp.min(jnp.where(tile == mt, it, _BIG), axis=1, keepdims=True)
            return merge(c, mt, cc)

        m, am = lax.fori_loop(tlo, thi, sbody, (m0, a0))
        acc = jnp.where(lane_k == k, am, acc)
        return acc, am

    acc, _ = lax.fori_loop(1, _K, kstep, (acc, am))
    idx_ref[...] = acc


def _knn(ptsT, b2d, bcol, tr, n):
    rb = 256 if n % 256 == 0 else n
    ct = 1024 if n % 1024 == 0 else n
    grid_spec = pltpu.PrefetchScalarGridSpec(
        num_scalar_prefetch=1,
        grid=(n // rb,),
        in_specs=[
            pl.BlockSpec((8, n), lambda i, *_: (0, 0)),
            pl.BlockSpec((1, n), lambda i, *_: (0, 0)),
            pl.BlockSpec((rb, 1), lambda i, *_: (i, 0)),
        ],
        out_specs=pl.BlockSpec((rb, _K), lambda i, *_: (i, 0)),
        scratch_shapes=[pltpu.VMEM((rb, n), jnp.float32)],
    )
    return pl.pallas_call(
        functools.partial(_knn_body, rb=rb, ct=ct, n=n),
        grid_spec=grid_spec,
        out_shape=jax.ShapeDtypeStruct((n, _K), jnp.int32),
    )(tr, ptsT, b2d, bcol)


def _tile_ranges(b, n, rb, ct):
    nblk = n // rb
    ntiles = n // ct
    bb = b.reshape(nblk, rb)
    bcast = b[None, :]
    lo_col = jnp.sum((bcast < bb[:, 0][:, None]).astype(jnp.int32), axis=1)
    hi_col = jnp.sum((bcast <= bb[:, -1][:, None]).astype(jnp.int32), axis=1)
    vals = jnp.arange(8, dtype=jnp.int32)
    hist = jnp.sum((bcast == vals[:, None]).astype(jnp.int32), axis=1)
    minsz = jnp.min(jnp.where(hist > 0, hist, n))
    tlo = jnp.where(minsz < _K, 0, lo_col // ct)
    thi = jnp.where(minsz < _K, ntiles, (hi_col + ct - 1) // ct)
    return jnp.stack([tlo, thi], axis=1).reshape(-1).astype(jnp.int32)


def _gmat_body(x_ref, pts_ref, feat_ref, cwT_ref, w3T_ref, l1bT_ref,
               g_ref, p2_ref, f3_ref):
    g_ref[...] = jnp.dot(x_ref[...], cwT_ref[...],
                         preferred_element_type=jnp.float32)
    p2_ref[...] = jnp.dot(pts_ref[...], w3T_ref[...],
                          preferred_element_type=jnp.float32)
    f3_ref[...] = jnp.dot(feat_ref[...], l1bT_ref[...],
                          preferred_element_type=jnp.float32)


def _gmat(x_cat, pts_pad, feature, cwT, w3T, l1bT, n, cin, inner):
    rbg = 512 if n % 512 == 0 else n
    c3 = cin + 3
    return pl.pallas_call(
        _gmat_body,
        grid=(n // rbg,),
        in_specs=[
            pl.BlockSpec((rbg, c3), lambda i: (i, 0)),
            pl.BlockSpec((rbg, 8), lambda i: (i, 0)),
            pl.BlockSpec((rbg, cin), lambda i: (i, 0)),
            pl.BlockSpec((c3, inner), lambda i: (0, 0)),
            pl.BlockSpec((8, inner), lambda i: (0, 0)),
            pl.BlockSpec((cin, inner), lambda i: (0, 0)),
        ],
        out_specs=[
            pl.BlockSpec((rbg, inner), lambda i: (i, 0)),
            pl.BlockSpec((rbg, inner), lambda i: (i, 0)),
            pl.BlockSpec((rbg, inner), lambda i: (i, 0)),
        ],
        out_shape=[
            jax.ShapeDtypeStruct((n, inner), jnp.float32),
            jax.ShapeDtypeStruct((n, inner), jnp.float32),
            jax.ShapeDtypeStruct((n, inner), jnp.float32),
        ],
    )(x_cat, pts_pad, feature, cwT, w3T, l1bT)


def _gather_rows(table, idx_flat):
    nrows = idx_flat.shape[0]
    d = table.shape[1]
    nw = 32
    per_w = nrows // nw
    ch = 128
    nch = per_w // ch
    mesh = plsc.VectorSubcoreMesh(core_axis_name="c", subcore_axis_name="s")

    @functools.partial(
        pl.kernel, mesh=mesh,
        out_type=jax.ShapeDtypeStruct((nrows, d), jnp.float32),
        scratch_types=[
            pltpu.VMEM((ch,), jnp.int32),
            pltpu.VMEM((ch, d), jnp.float32),
            pltpu.VMEM((ch,), jnp.int32),
            pltpu.VMEM((ch, d), jnp.float32),
            pltpu.SemaphoreType.DMA,
            pltpu.SemaphoreType.DMA,
        ],
    )
    def gk(table_hbm, idx_hbm, out_hbm, idx0, rows0, idx1, rows1, sem0, sem1):
        wid = lax.axis_index("s") * 2 + lax.axis_index("c")
        base = wid * per_w
        idx_v = [idx0, idx1]
        rows_v = [rows0, rows1]
        sems = [sem0, sem1]

        def start(c, slot):
            off = base + c * ch
            pltpu.sync_copy(idx_hbm.at[pl.ds(off, ch)], idx_v[slot])
            pltpu.async_copy(table_hbm.at[idx_v[slot]], rows_v[slot], sems[slot])

        def drain(c, slot):
            off = base + c * ch
            pltpu.make_async_copy(table_hbm.at[idx_v[slot]], rows_v[slot],
                                  sems[slot]).wait()
            pltpu.sync_copy(rows_v[slot], out_hbm.at[pl.ds(off, ch)])

        start(0, 0)
        def body(c, carry):
            slot = lax.rem(c, 2)
            nslot = 1 - slot
            @pl.when(c + 1 < nch)
            def _():
                jax.lax.switch(nslot, [lambda: start(c + 1, 0),
                                       lambda: start(c + 1, 1)])
            jax.lax.switch(slot, [lambda: drain(c, 0), lambda: drain(c, 1)])
            return carry

        lax.fori_loop(0, nch, body, 0)

    return gk(table, idx_flat)


def _stats1_body(h_ref, p2_ref, ms_ref, out_ref, *, inner):
    i = pl.program_id(0)
    h = h_ref[...]
    p2 = p2_ref[...]
    hs = jnp.dot(ms_ref[...], h, preferred_element_type=jnp.float32)
    s1 = jnp.sum(h, axis=0, keepdims=True)
    s2 = jnp.sum(h * h, axis=0, keepdims=True)
    s3 = jnp.sum(p2 * hs, axis=0, keepdims=True)
    s4 = jnp.sum(p2, axis=0, keepdims=True)
    s5 = jnp.sum(p2 * p2, axis=0, keepdims=True)
    contrib = jnp.concatenate(
        [s1, s2, s3, s4, s5, jnp.zeros((3, inner), jnp.float32)], axis=0)

    @pl.when(i == 0)
    def _():
        out_ref[...] = jnp.zeros_like(out_ref)

    out_ref[...] += contrib


def _stats1(h, p2, msum, n, inner):
    pb = 128 if n % 128 == 0 else n
    return pl.pallas_call(
        functools.partial(_stats1_body, inner=inner),
        grid=(n // pb,),
        in_specs=[
            pl.BlockSpec((pb * _K, inner), lambda i: (i, 0)),
            pl.BlockSpec((pb, inner), lambda i: (i, 0)),
            pl.BlockSpec((pb, pb * _K), lambda i: (0, 0)),
        ],
        out_specs=pl.BlockSpec((8, inner), lambda i: (0, 0)),
        out_shape=jax.ShapeDtypeStruct((8, inner), jnp.float32),
    )(h, p2, msum)


def _passb_body(h_ref, p2_ref, a_ref, c_ref, y_ref, w_ref, *, pb, inner):
    a = a_ref[...]
    u = c_ref[...] - a * p2_ref[...]
    s = a * h_ref[:, 0, :] + u
    y = None
    wcols = []
    for k in range(_K):
        xk = a * h_ref[:, k, :] + u
        wk = jnp.sum(xk * s, axis=1, keepdims=True)
        wcols.append(wk)
        yk = xk * wk
        y = yk if y is None else y + yk
    w_ref[...] = jnp.concatenate(wcols, axis=1)
    y_ref[...] = y


def _passb(h3, p2, a1, c1, n, inner):
    pb = 128 if n % 128 == 0 else n
    return pl.pallas_call(
        functools.partial(_passb_body, pb=pb, inner=inner),
        grid=(n // pb,),
        in_specs=[
            pl.BlockSpec((pb, _K, inner), lambda i: (i, 0, 0)),
            pl.BlockSpec((pb, inner), lambda i: (i, 0)),
            pl.BlockSpec((1, inner), lambda i: (0, 0)),
            pl.BlockSpec((1, inner), lambda i: (0, 0)),
        ],
        out_specs=[
            pl.BlockSpec((pb, inner), lambda i: (i, 0)),
            pl.BlockSpec((pb, _K), lambda i: (i, 0)),
        ],
        out_shape=[
            jax.ShapeDtypeStruct((n, inner), jnp.float32),
            jax.ShapeDtypeStruct((n, _K), jnp.float32),
        ],
    )(h3, p2, a1, c1)


def _passc_body(h2_ref, wr_ref, ms_ref, z_ref, acc_ref, *, inner):
    i = pl.program_id(0)
    hw = h2_ref[...] * wr_ref[...]
    z = jnp.dot(ms_ref[...], hw, preferred_element_type=jnp.float32)
    z_ref[...] = z
    contrib = jnp.concatenate(
        [jnp.sum(z, axis=0, keepdims=True),
         jnp.sum(z * z, axis=0, keepdims=True),
         jnp.zeros((6, inner), jnp.float32)], axis=0)

    @pl.when(i == 0)
    def _():
        acc_ref[...] = jnp.zeros_like(acc_ref)

    acc_ref[...] += contrib


def _passc(h2, wr, msum, n, inner):
    pb = 128 if n % 128 == 0 else n
    return pl.pallas_call(
        functools.partial(_passc_body, inner=inner),
        grid=(n // pb,),
        in_specs=[
            pl.BlockSpec((pb * _K, inner), lambda i: (i, 0)),
            pl.BlockSpec((pb * _K, 1), lambda i: (i, 0)),
            pl.BlockSpec((pb, pb * _K), lambda i: (0, 0)),
        ],
        out_specs=[
            pl.BlockSpec((pb, inner), lambda i: (i, 0)),
            pl.BlockSpec((8, inner), lambda i: (0, 0)),
        ],
        out_shape=[
            jax.ShapeDtypeStruct((n, inner), jnp.float32),
            jax.ShapeDtypeStruct((8, inner), jnp.float32),
        ],
    )(h2, wr, msum)


def _passd1_body(z_ref, f3_ref, a2_ref, c2_ref, l1aT_ref, b1_ref,
                 t_ref, acc_ref, *, cin):
    i = pl.program_id(0)
    r = jnp.maximum(a2_ref[...] * z_ref[...] + c2_ref[...], 0.0)
    t = (jnp.dot(r, l1aT_ref[...], preferred_element_type=jnp.float32)
         + f3_ref[...] + b1_ref[...])
    t_ref[...] = t
    contrib = jnp.concatenate(
        [jnp.sum(t, axis=0, keepdims=True),
         jnp.sum(t * t, axis=0, keepdims=True),
         jnp.zeros((6, cin), jnp.float32)], axis=0)

    @pl.when(i == 0)
    def _():
        acc_ref[...] = jnp.zeros_like(acc_ref)

    acc_ref[...] += contrib


def _passd1(z, f3, a2, c2, l1aT, b1, n, cin, inner):
    rbg = 512 if n % 512 == 0 else n
    return pl.pallas_call(
        functools.partial(_passd1_body, cin=cin),
        grid=(n // rbg,),
        in_specs=[
            pl.BlockSpec((rbg, inner), lambda i: (i, 0)),
            pl.BlockSpec((rbg, cin), lambda i: (i, 0)),
            pl.BlockSpec((1, inner), lambda i: (0, 0)),
            pl.BlockSpec((1, inner), lambda i: (0, 0)),
            pl.BlockSpec((inner, cin), lambda i: (0, 0)),
            pl.BlockSpec((1, cin), lambda i: (0, 0)),
        ],
        out_specs=[
            pl.BlockSpec((rbg, cin), lambda i: (i, 0)),
            pl.BlockSpec((8, cin), lambda i: (0, 0)),
        ],
        out_shape=[
            jax.ShapeDtypeStruct((n, cin), jnp.float32),
            jax.ShapeDtypeStruct((8, cin), jnp.float32),
        ],
    )(z, f3, a2, c2, l1aT, b1)


def _passd2_body(t_ref, a3_ref, c3_ref, l2T_ref, b2_ref, o_ref):
    r = jnp.maximum(a3_ref[...] * t_ref[...] + c3_ref[...], 0.0)
    o_ref[...] = (jnp.dot(r, l2T_ref[...], preferred_element_type=jnp.float32)
                  + b2_ref[...])


def _passd2(t, a3, c3, l2T, b2, n, cin):
    rbg = 512 if n % 512 == 0 else n
    return pl.pallas_call(
        _passd2_body,
        grid=(n // rbg,),
        in_specs=[
            pl.BlockSpec((rbg, cin), lambda i: (i, 0)),
            pl.BlockSpec((1, cin), lambda i: (0, 0)),
            pl.BlockSpec((1, cin), lambda i: (0, 0)),
            pl.BlockSpec((cin, cin), lambda i: (0, 0)),
            pl.BlockSpec((1, cin), lambda i: (0, 0)),
        ],
        out_specs=pl.BlockSpec((rbg, cin), lambda i: (i, 0)),
        out_shape=jax.ShapeDtypeStruct((n, cin), jnp.float32),
    )(t, a3, c3, l2T, b2)


def kernel(coords, points, feature, conv_w, bn1_g, bn1_b, bn2_g, bn2_b,
           lin1_w, lin1_b, bn3_g, bn3_b, lin2_w, lin2_b):
    n, cin = feature.shape
    inner = conv_w.shape[0]
    f32 = jnp.float32
    b = coords[:, 3].astype(jnp.int32)

    ptsT = jnp.concatenate([points.T, jnp.zeros((5, n), f32)], axis=0)
    rb = 256 if n % 256 == 0 else n
    ct = 1024 if n % 1024 == 0 else n
    tr = _tile_ranges(b, n, rb, ct)
    idx = _knn(ptsT, b[None, :], b[:, None], tr, n)

    x_cat = jnp.concatenate([feature, points], axis=1)
    pts_pad = jnp.concatenate([points, jnp.zeros((n, 5), f32)], axis=1)
    w3T = jnp.concatenate(
        [conv_w[:, cin:].T, jnp.zeros((5, inner), f32)], axis=0)
    g, p2, f3 = _gmat(x_cat, pts_pad, feature, conv_w.T, w3T,
                      lin1_w[:, inner:].T, n, cin, inner)

    idx_flat = idx.reshape(-1)
    pb = 128 if n % 128 == 0 else n
    msum = (jnp.arange(pb, dtype=jnp.int32)[:, None]
            == (jnp.arange(pb * _K, dtype=jnp.int32)[None, :] // _K)
            ).astype(f32)
    h = _gather_rows(g, idx_flat)
    h3 = h.reshape(n, _K, inner)
    s = _stats1(h, p2, msum, n, inner)
    nk = jnp.float32(n * _K)
    mean1 = (s[0] - _K * s[3]) / nk
    ex2 = (s[1] - 2.0 * s[2] + _K * s[4]) / nk
    var1 = ex2 - mean1 * mean1
    a1 = bn1_g / jnp.sqrt(var1 + _EPS)
    c1 = bn1_b - a1 * mean1

    y, w = _passb(h3, p2, a1[None], c1[None], n, inner)

    h2 = _gather_rows(y, idx_flat)
    wr = w.reshape(n * _K, 1)
    z, acc2 = _passc(h2, wr, msum, n, inner)
    mean2 = acc2[0] / n
    var2 = acc2[1] / n - mean2 * mean2
    a2 = bn2_g / jnp.sqrt(var2 + _EPS)
    c2 = bn2_b - a2 * mean2

    t, acc3 = _passd1(z, f3, a2[None], c2[None], lin1_w[:, :inner].T,
                      lin1_b[None], n, cin, inner)
    mean3 = acc3[0] / n
    var3 = acc3[1] / n - mean3 * mean3
    a3 = bn3_g / jnp.sqrt(var3 + _EPS)
    c3 = bn3_b - a3 * mean3

    return _passd2(t, a3[None], c3[None], lin2_w.T, lin2_b[None], n, cin)


# speedup vs baseline: 1.3142x; 1.0280x over previous
import functools

import jax
import jax.numpy as jnp
from jax import lax
from jax.experimental import pallas as pl
from jax.experimental.pallas import tpu as pltpu
from jax.experimental.pallas import tpu_sc as plsc

_EPS = 1e-5
_K = 16
_BIG = 2**30


def _knn_body(tr_ref, ptsT_ref, b2d_ref, bcol_ref, idx_ref, d2_ref,
              *, rb, ct, n, i0):
    i = pl.program_id(0) + i0
    tlo = tr_ref[2 * i]
    thi = tr_ref[2 * i + 1]
    prow = ptsT_ref[:, pl.ds(i * rb, rb)]
    sqrow = lax.dot_general(
        prow * prow, jnp.ones((8, 1), jnp.float32),
        (((0,), (0,)), ((), ())), preferred_element_type=jnp.float32)
    bcol = bcol_ref[...]
    m0 = jnp.full((rb, 1), jnp.inf, jnp.float32)
    a0 = jnp.full((rb, 1), _BIG, jnp.int32)
    iota_c = lax.broadcasted_iota(jnp.int32, (rb, ct), 1)
    lane_k = lax.broadcasted_iota(jnp.int32, (rb, _K), 1)

    def merge(carry, mt, cc):
        m, am = carry
        am2 = jnp.where(mt < m, cc,
                        jnp.where(mt == m, jnp.minimum(am, cc), am))
        return jnp.minimum(m, mt), am2

    def init_body(t, carry):
        off = pl.multiple_of(t * ct, ct)
        ptile = ptsT_ref[:, pl.ds(off, ct)]
        pp = lax.dot_general(prow, ptile, (((0,), (0,)), ((), ())),
                             preferred_element_type=jnp.float32)
        sqcol = jnp.sum(ptile * ptile, axis=0, keepdims=True)
        d2 = sqrow + sqcol - 2.0 * pp
        same = bcol == b2d_ref[:, pl.ds(off, ct)]
        tile = jnp.where(same, d2, jnp.inf)
        d2_ref[:, pl.ds(off, ct)] = tile
        it = iota_c + t * ct
        mt = jnp.min(tile, axis=1, keepdims=True)
        cc = jnp.min(jnp.where(tile == mt, it, _BIG), axis=1, keepdims=True)
        return merge(carry, mt, cc)

    m, am = lax.fori_loop(tlo, thi, init_body, (m0, a0))
    acc = jnp.where(lane_k == 0, am, 0)

    def kstep(k, carry):
        acc, aprev = carry

        def sbody(t, c):
            off = pl.multiple_of(t * ct, ct)
            tile = d2_ref[:, pl.ds(off, ct)]
            it = iota_c + t * ct
            tile = jnp.where(it == aprev, jnp.inf, tile)
            d2_ref[:, pl.ds(off, ct)] = tile
            mt = jnp.min(tile, axis=1, keepdims=True)
            cc = jnp.min(jnp.where(tile == mt, it, _BIG), axis=1, keepdims=True)
            return merge(c, mt, cc)

        m, am = lax.fori_loop(tlo, thi, sbody, (m0, a0))
        acc = jnp.where(lane_k == k, am, acc)
        return acc, am

    acc, _ = lax.fori_loop(1, _K, kstep, (acc, am))
    idx_ref[...] = acc


def _knn(ptsT, b2d, bcol, tr, n, i0, nout):
    rb = 256 if n % 256 == 0 else n
    ct = 1024 if n % 1024 == 0 else n
    grid_spec = pltpu.PrefetchScalarGridSpec(
        num_scalar_prefetch=1,
        grid=(nout // rb,),
        in_specs=[
            pl.BlockSpec((8, n), lambda i, *_: (0, 0)),
            pl.BlockSpec((1, n), lambda i, *_: (0, 0)),
            pl.BlockSpec((rb, 1), lambda i, *_: (i + i0, 0)),
        ],
        out_specs=pl.BlockSpec((rb, _K), lambda i, *_: (i, 0)),
        scratch_shapes=[pltpu.VMEM((rb, n), jnp.float32)],
    )
    return pl.pallas_call(
        functools.partial(_knn_body, rb=rb, ct=ct, n=n, i0=i0),
        grid_spec=grid_spec,
        out_shape=jax.ShapeDtypeStruct((nout, _K), jnp.int32),
    )(tr, ptsT, b2d, bcol)


def _tile_ranges(b, n, rb, ct):
    nblk = n // rb
    ntiles = n // ct
    bb = b.reshape(nblk, rb)
    bcast = b[None, :]
    lo_col = jnp.sum((bcast < bb[:, 0][:, None]).astype(jnp.int32), axis=1)
    hi_col = jnp.sum((bcast <= bb[:, -1][:, None]).astype(jnp.int32), axis=1)
    vals = jnp.arange(8, dtype=jnp.int32)
    hist = jnp.sum((bcast == vals[:, None]).astype(jnp.int32), axis=1)
    minsz = jnp.min(jnp.where(hist > 0, hist, n))
    tlo = jnp.where(minsz < _K, 0, lo_col // ct)
    thi = jnp.where(minsz < _K, ntiles, (hi_col + ct - 1) // ct)
    return jnp.stack([tlo, thi], axis=1).reshape(-1).astype(jnp.int32)


def _gmat_body(x_ref, pts_ref, feat_ref, cwT_ref, w3T_ref, l1bT_ref,
               g_ref, p2_ref, f3_ref):
    g_ref[...] = jnp.dot(x_ref[...], cwT_ref[...],
                         preferred_element_type=jnp.float32)
    p2_ref[...] = jnp.dot(pts_ref[...], w3T_ref[...],
                          preferred_element_type=jnp.float32)
    f3_ref[...] = jnp.dot(feat_ref[...], l1bT_ref[...],
                          preferred_element_type=jnp.float32)


def _gmat(x_cat, pts_pad, feature, cwT, w3T, l1bT, n, cin, inner):
    rbg = 512 if n % 512 == 0 else n
    c3 = cin + 3
    return pl.pallas_call(
        _gmat_body,
        grid=(n // rbg,),
        in_specs=[
            pl.BlockSpec((rbg, c3), lambda i: (i, 0)),
            pl.BlockSpec((rbg, 8), lambda i: (i, 0)),
            pl.BlockSpec((rbg, cin), lambda i: (i, 0)),
            pl.BlockSpec((c3, inner), lambda i: (0, 0)),
            pl.BlockSpec((8, inner), lambda i: (0, 0)),
            pl.BlockSpec((cin, inner), lambda i: (0, 0)),
        ],
        out_specs=[
            pl.BlockSpec((rbg, inner), lambda i: (i, 0)),
            pl.BlockSpec((rbg, inner), lambda i: (i, 0)),
            pl.BlockSpec((rbg, inner), lambda i: (i, 0)),
        ],
        out_shape=[
            jax.ShapeDtypeStruct((n, inner), jnp.float32),
            jax.ShapeDtypeStruct((n, inner), jnp.float32),
            jax.ShapeDtypeStruct((n, inner), jnp.float32),
        ],
    )(x_cat, pts_pad, feature, cwT, w3T, l1bT)


def _gather_rows(table, idx_flat):
    nrows = idx_flat.shape[0]
    d = table.shape[1]
    nw = 32
    per_w = nrows // nw
    ch = 128
    nch = per_w // ch
    mesh = plsc.VectorSubcoreMesh(core_axis_name="c", subcore_axis_name="s")

    @functools.partial(
        pl.kernel, mesh=mesh,
        out_type=jax.ShapeDtypeStruct((nrows, d), jnp.float32),
        scratch_types=[
            pltpu.VMEM((ch,), jnp.int32),
            pltpu.VMEM((ch, d), jnp.float32),
            pltpu.VMEM((ch,), jnp.int32),
            pltpu.VMEM((ch, d), jnp.float32),
            pltpu.SemaphoreType.DMA,
            pltpu.SemaphoreType.DMA,
        ],
    )
    def gk(table_hbm, idx_hbm, out_hbm, idx0, rows0, idx1, rows1, sem0, sem1):
        wid = lax.axis_index("s") * 2 + lax.axis_index("c")
        base = wid * per_w
        idx_v = [idx0, idx1]
        rows_v = [rows0, rows1]
        sems = [sem0, sem1]

        def start(c, slot):
            off = base + c * ch
            pltpu.sync_copy(idx_hbm.at[pl.ds(off, ch)], idx_v[slot])
            pltpu.async_copy(table_hbm.at[idx_v[slot]], rows_v[slot], sems[slot])

        def drain(c, slot):
            off = base + c * ch
            pltpu.make_async_copy(table_hbm.at[idx_v[slot]], rows_v[slot],
                                  sems[slot]).wait()
            pltpu.sync_copy(rows_v[slot], out_hbm.at[pl.ds(off, ch)])

        start(0, 0)
        def body(c, carry):
            slot = lax.rem(c, 2)
            nslot = 1 - slot
            @pl.when(c + 1 < nch)
            def _():
                jax.lax.switch(nslot, [lambda: start(c + 1, 0),
                                       lambda: start(c + 1, 1)])
            jax.lax.switch(slot, [lambda: drain(c, 0), lambda: drain(c, 1)])
            return carry

        lax.fori_loop(0, nch, body, 0)

    return gk(table, idx_flat)


def _stats1_body(h_ref, p2_ref, ms_ref, out_ref, *, inner):
    i = pl.program_id(0)
    h = h_ref[...]
    p2 = p2_ref[...]
    hs = jnp.dot(ms_ref[...], h, preferred_element_type=jnp.float32)
    s1 = jnp.sum(h, axis=0, keepdims=True)
    s2 = jnp.sum(h * h, axis=0, keepdims=True)
    s3 = jnp.sum(p2 * hs, axis=0, keepdims=True)
    s4 = jnp.sum(p2, axis=0, keepdims=True)
    s5 = jnp.sum(p2 * p2, axis=0, keepdims=True)
    contrib = jnp.concatenate(
        [s1, s2, s3, s4, s5, jnp.zeros((3, inner), jnp.float32)], axis=0)

    @pl.when(i == 0)
    def _():
        out_ref[...] = jnp.zeros_like(out_ref)

    out_ref[...] += contrib


def _stats1(h, p2, msum, n, inner):
    pb = 128 if n % 128 == 0 else n
    return pl.pallas_call(
        functools.partial(_stats1_body, inner=inner),
        grid=(n // pb,),
        in_specs=[
            pl.BlockSpec((pb * _K, inner), lambda i: (i, 0)),
            pl.BlockSpec((pb, inner), lambda i: (i, 0)),
            pl.BlockSpec((pb, pb * _K), lambda i: (0, 0)),
        ],
        out_specs=pl.BlockSpec((8, inner), lambda i: (0, 0)),
        out_shape=jax.ShapeDtypeStruct((8, inner), jnp.float32),
    )(h, p2, msum)


def _passb_body(h_ref, p2_ref, a_ref, c_ref, y_ref, w_ref, *, pb, inner):
    a = a_ref[...]
    u = c_ref[...] - a * p2_ref[...]
    s = a * h_ref[:, 0, :] + u
    y = None
    wcols = []
    for k in range(_K):
        xk = a * h_ref[:, k, :] + u
        wk = jnp.sum(xk * s, axis=1, keepdims=True)
        wcols.append(wk)
        yk = xk * wk
        y = yk if y is None else y + yk
    w_ref[...] = jnp.concatenate(wcols, axis=1)
    y_ref[...] = y


def _passb(h3, p2, a1, c1, n, inner):
    pb = 128 if n % 128 == 0 else n
    return pl.pallas_call(
        functools.partial(_passb_body, pb=pb, inner=inner),
        grid=(n // pb,),
        in_specs=[
            pl.BlockSpec((pb, _K, inner), lambda i: (i, 0, 0)),
            pl.BlockSpec((pb, inner), lambda i: (i, 0)),
            pl.BlockSpec((1, inner), lambda i: (0, 0)),
            pl.BlockSpec((1, inner), lambda i: (0, 0)),
        ],
        out_specs=[
            pl.BlockSpec((pb, inner), lambda i: (i, 0)),
            pl.BlockSpec((pb, _K), lambda i: (i, 0)),
        ],
        out_shape=[
            jax.ShapeDtypeStruct((n, inner), jnp.float32),
            jax.ShapeDtypeStruct((n, _K), jnp.float32),
        ],
    )(h3, p2, a1, c1)


def _passc_body(h2_ref, wr_ref, ms_ref, z_ref, acc_ref, *, inner):
    i = pl.program_id(0)
    hw = h2_ref[...] * wr_ref[...]
    z = jnp.dot(ms_ref[...], hw, preferred_element_type=jnp.float32)
    z_ref[...] = z
    contrib = jnp.concatenate(
        [jnp.sum(z, axis=0, keepdims=True),
         jnp.sum(z * z, axis=0, keepdims=True),
         jnp.zeros((6, inner), jnp.float32)], axis=0)

    @pl.when(i == 0)
    def _():
        acc_ref[...] = jnp.zeros_like(acc_ref)

    acc_ref[...] += contrib


def _passc(h2, wr, msum, n, inner):
    pb = 128 if n % 128 == 0 else n
    return pl.pallas_call(
        functools.partial(_passc_body, inner=inner),
        grid=(n // pb,),
        in_specs=[
            pl.BlockSpec((pb * _K, inner), lambda i: (i, 0)),
            pl.BlockSpec((pb * _K, 1), lambda i: (i, 0)),
            pl.BlockSpec((pb, pb * _K), lambda i: (0, 0)),
        ],
        out_specs=[
            pl.BlockSpec((pb, inner), lambda i: (i, 0)),
            pl.BlockSpec((8, inner), lambda i: (0, 0)),
        ],
        out_shape=[
            jax.ShapeDtypeStruct((n, inner), jnp.float32),
            jax.ShapeDtypeStruct((8, inner), jnp.float32),
        ],
    )(h2, wr, msum)


def _passd1_body(z_ref, f3_ref, a2_ref, c2_ref, l1aT_ref, b1_ref,
                 t_ref, acc_ref, *, cin):
    i = pl.program_id(0)
    r = jnp.maximum(a2_ref[...] * z_ref[...] + c2_ref[...], 0.0)
    t = (jnp.dot(r, l1aT_ref[...], preferred_element_type=jnp.float32)
         + f3_ref[...] + b1_ref[...])
    t_ref[...] = t
    contrib = jnp.concatenate(
        [jnp.sum(t, axis=0, keepdims=True),
         jnp.sum(t * t, axis=0, keepdims=True),
         jnp.zeros((6, cin), jnp.float32)], axis=0)

    @pl.when(i == 0)
    def _():
        acc_ref[...] = jnp.zeros_like(acc_ref)

    acc_ref[...] += contrib


def _passd1(z, f3, a2, c2, l1aT, b1, n, cin, inner):
    rbg = 512 if n % 512 == 0 else n
    return pl.pallas_call(
        functools.partial(_passd1_body, cin=cin),
        grid=(n // rbg,),
        in_specs=[
            pl.BlockSpec((rbg, inner), lambda i: (i, 0)),
            pl.BlockSpec((rbg, cin), lambda i: (i, 0)),
            pl.BlockSpec((1, inner), lambda i: (0, 0)),
            pl.BlockSpec((1, inner), lambda i: (0, 0)),
            pl.BlockSpec((inner, cin), lambda i: (0, 0)),
            pl.BlockSpec((1, cin), lambda i: (0, 0)),
        ],
        out_specs=[
            pl.BlockSpec((rbg, cin), lambda i: (i, 0)),
            pl.BlockSpec((8, cin), lambda i: (0, 0)),
        ],
        out_shape=[
            jax.ShapeDtypeStruct((n, cin), jnp.float32),
            jax.ShapeDtypeStruct((8, cin), jnp.float32),
        ],
    )(z, f3, a2, c2, l1aT, b1)


def _passd2_body(t_ref, a3_ref, c3_ref, l2T_ref, b2_ref, o_ref):
    r = jnp.maximum(a3_ref[...] * t_ref[...] + c3_ref[...], 0.0)
    o_ref[...] = (jnp.dot(r, l2T_ref[...], preferred_element_type=jnp.float32)
                  + b2_ref[...])


def _passd2(t, a3, c3, l2T, b2, n, cin):
    rbg = 512 if n % 512 == 0 else n
    return pl.pallas_call(
        _passd2_body,
        grid=(n // rbg,),
        in_specs=[
            pl.BlockSpec((rbg, cin), lambda i: (i, 0)),
            pl.BlockSpec((1, cin), lambda i: (0, 0)),
            pl.BlockSpec((1, cin), lambda i: (0, 0)),
            pl.BlockSpec((cin, cin), lambda i: (0, 0)),
            pl.BlockSpec((1, cin), lambda i: (0, 0)),
        ],
        out_specs=pl.BlockSpec((rbg, cin), lambda i: (i, 0)),
        out_shape=jax.ShapeDtypeStruct((n, cin), jnp.float32),
    )(t, a3, c3, l2T, b2)


def kernel(coords, points, feature, conv_w, bn1_g, bn1_b, bn2_g, bn2_b,
           lin1_w, lin1_b, bn3_g, bn3_b, lin2_w, lin2_b):
    n, cin = feature.shape
    inner = conv_w.shape[0]
    f32 = jnp.float32
    b = coords[:, 3].astype(jnp.int32)

    ptsT = jnp.concatenate([points.T, jnp.zeros((5, n), f32)], axis=0)
    rb = 256 if n % 256 == 0 else n
    ct = 1024 if n % 1024 == 0 else n
    tr = _tile_ranges(b, n, rb, ct)
    nh = n // 2
    b2d = b[None, :]
    bcol = b[:, None]
    idx_lo = _knn(ptsT, b2d, bcol, tr, n, 0, nh)
    idx_hi = _knn(ptsT, b2d, bcol, tr, n, nh // rb, nh)

    x_cat = jnp.concatenate([feature, points], axis=1)
    pts_pad = jnp.concatenate([points, jnp.zeros((n, 5), f32)], axis=1)
    w3T = jnp.concatenate(
        [conv_w[:, cin:].T, jnp.zeros((5, inner), f32)], axis=0)
    g, p2, f3 = _gmat(x_cat, pts_pad, feature, conv_w.T, w3T,
                      lin1_w[:, inner:].T, n, cin, inner)

    pb = 128 if n % 128 == 0 else n
    msum = (jnp.arange(pb, dtype=jnp.int32)[:, None]
            == (jnp.arange(pb * _K, dtype=jnp.int32)[None, :] // _K)
            ).astype(f32)
    h_lo = _gather_rows(g, idx_lo.reshape(-1))
    h_hi = _gather_rows(g, idx_hi.reshape(-1))
    p2_lo, p2_hi = p2[:nh], p2[nh:]
    s = (_stats1(h_lo, p2_lo, msum, nh, inner)
         + _stats1(h_hi, p2_hi, msum, nh, inner))
    nk = jnp.float32(n * _K)
    mean1 = (s[0] - _K * s[3]) / nk
    ex2 = (s[1] - 2.0 * s[2] + _K * s[4]) / nk
    var1 = ex2 - mean1 * mean1
    a1 = bn1_g / jnp.sqrt(var1 + _EPS)
    c1 = bn1_b - a1 * mean1

    y_lo, w_lo = _passb(h_lo.reshape(nh, _K, inner), p2_lo,
                        a1[None], c1[None], nh, inner)
    y_hi, w_hi = _passb(h_hi.reshape(nh, _K, inner), p2_hi,
                        a1[None], c1[None], nh, inner)
    y = jnp.concatenate([y_lo, y_hi], axis=0)

    h2_lo = _gather_rows(y, idx_lo.reshape(-1))
    h2_hi = _gather_rows(y, idx_hi.reshape(-1))
    z_lo, acc2_lo = _passc(h2_lo, w_lo.reshape(nh * _K, 1), msum, nh, inner)
    z_hi, acc2_hi = _passc(h2_hi, w_hi.reshape(nh * _K, 1), msum, nh, inner)
    z = jnp.concatenate([z_lo, z_hi], axis=0)
    acc2 = acc2_lo + acc2_hi
    mean2 = acc2[0] / n
    var2 = acc2[1] / n - mean2 * mean2
    a2 = bn2_g / jnp.sqrt(var2 + _EPS)
    c2 = bn2_b - a2 * mean2

    t, acc3 = _passd1(z, f3, a2[None], c2[None], lin1_w[:, :inner].T,
                      lin1_b[None], n, cin, inner)
    mean3 = acc3[0] / n
    var3 = acc3[1] / n - mean3 * mean3
    a3 = bn3_g / jnp.sqrt(var3 + _EPS)
    c3 = bn3_b - a3 * mean3

    return _passd2(t, a3[None], c3[None], lin2_w.T, lin2_b[None], n, cin)


# speedup vs baseline: 1.3145x; 1.0002x over previous
import functools

import jax
import jax.numpy as jnp
from jax import lax
from jax.experimental import pallas as pl
from jax.experimental.pallas import tpu as pltpu
from jax.experimental.pallas import tpu_sc as plsc

_EPS = 1e-5
_K = 16
_BIG = 2**30


def _knn_body(tr_ref, ptsT_ref, b2d_ref, bcol_ref, idx_ref, d2_ref,
              *, rb, ct, n, i0):
    i = pl.program_id(0) + i0
    tlo = tr_ref[2 * i]
    thi = tr_ref[2 * i + 1]
    prow = ptsT_ref[:, pl.ds(i * rb, rb)]
    sqrow = lax.dot_general(
        prow * prow, jnp.ones((8, 1), jnp.float32),
        (((0,), (0,)), ((), ())), preferred_element_type=jnp.float32)
    bcol = bcol_ref[...]
    m0 = jnp.full((rb, 1), jnp.inf, jnp.float32)
    a0 = jnp.full((rb, 1), _BIG, jnp.int32)
    iota_c = lax.broadcasted_iota(jnp.int32, (rb, ct), 1)
    lane_k = lax.broadcasted_iota(jnp.int32, (rb, _K), 1)

    def merge(carry, mt, cc):
        m, am = carry
        am2 = jnp.where(mt < m, cc,
                        jnp.where(mt == m, jnp.minimum(am, cc), am))
        return jnp.minimum(m, mt), am2

    def init_body(t, carry):
        off = pl.multiple_of(t * ct, ct)
        ptile = ptsT_ref[:, pl.ds(off, ct)]
        pp = lax.dot_general(prow, ptile, (((0,), (0,)), ((), ())),
                             preferred_element_type=jnp.float32)
        sqcol = jnp.sum(ptile * ptile, axis=0, keepdims=True)
        d2 = sqrow + sqcol - 2.0 * pp
        same = bcol == b2d_ref[:, pl.ds(off, ct)]
        tile = jnp.where(same, d2, jnp.inf)
        d2_ref[:, pl.ds(off, ct)] = tile
        it = iota_c + t * ct
        mt = jnp.min(tile, axis=1, keepdims=True)
        cc = jnp.min(jnp.where(tile == mt, it, _BIG), axis=1, keepdims=True)
        return merge(carry, mt, cc)

    m, am = lax.fori_loop(tlo, thi, init_body, (m0, a0))
    acc = jnp.where(lane_k == 0, am, 0)

    def kstep(k, carry):
        acc, aprev = carry

        def sbody(t, c):
            off = pl.multiple_of(t * ct, ct)
            tile = d2_ref[:, pl.ds(off, ct)]
            it = iota_c + t * ct
            tile = jnp.where(it == aprev, jnp.inf, tile)
            d2_ref[:, pl.ds(off, ct)] = tile
            mt = jnp.min(tile, axis=1, keepdims=True)
            cc = jnp.min(jnp.where(tile == mt, it, _BIG), axis=1, keepdims=True)
            return merge(c, mt, cc)

        m, am = lax.fori_loop(tlo, thi, sbody, (m0, a0))
        acc = jnp.where(lane_k == k, am, acc)
        return acc, am

    acc, _ = lax.fori_loop(1, _K, kstep, (acc, am))
    idx_ref[...] = acc


def _knn(ptsT, b2d, bcol, tr, n, i0, nout):
    rb = 256 if n % 256 == 0 else n
    ct = 1024 if n % 1024 == 0 else n
    grid_spec = pltpu.PrefetchScalarGridSpec(
        num_scalar_prefetch=1,
        grid=(nout // rb,),
        in_specs=[
            pl.BlockSpec((8, n), lambda i, *_: (0, 0)),
            pl.BlockSpec((1, n), lambda i, *_: (0, 0)),
            pl.BlockSpec((rb, 1), lambda i, *_: (i + i0, 0)),
        ],
        out_specs=pl.BlockSpec((rb, _K), lambda i, *_: (i, 0)),
        scratch_shapes=[pltpu.VMEM((rb, n), jnp.float32)],
    )
    return pl.pallas_call(
        functools.partial(_knn_body, rb=rb, ct=ct, n=n, i0=i0),
        grid_spec=grid_spec,
        out_shape=jax.ShapeDtypeStruct((nout, _K), jnp.int32),
    )(tr, ptsT, b2d, bcol)


def _tile_ranges(b, n, rb, ct):
    nblk = n // rb
    ntiles = n // ct
    bb = b.reshape(nblk, rb)
    bcast = b[None, :]
    lo_col = jnp.sum((bcast < bb[:, 0][:, None]).astype(jnp.int32), axis=1)
    hi_col = jnp.sum((bcast <= bb[:, -1][:, None]).astype(jnp.int32), axis=1)
    vals = jnp.arange(8, dtype=jnp.int32)
    hist = jnp.sum((bcast == vals[:, None]).astype(jnp.int32), axis=1)
    minsz = jnp.min(jnp.where(hist > 0, hist, n))
    tlo = jnp.where(minsz < _K, 0, lo_col // ct)
    thi = jnp.where(minsz < _K, ntiles, (hi_col + ct - 1) // ct)
    return jnp.stack([tlo, thi], axis=1).reshape(-1).astype(jnp.int32)


def _gmat_body(x_ref, pts_ref, feat_ref, cwT_ref, w3T_ref, l1bT_ref,
               g_ref, p2_ref, f3_ref):
    g_ref[...] = jnp.dot(x_ref[...], cwT_ref[...],
                         preferred_element_type=jnp.float32)
    p2_ref[...] = jnp.dot(pts_ref[...], w3T_ref[...],
                          preferred_element_type=jnp.float32)
    f3_ref[...] = jnp.dot(feat_ref[...], l1bT_ref[...],
                          preferred_element_type=jnp.float32)


def _gmat(x_cat, pts_pad, feature, cwT, w3T, l1bT, n, cin, inner):
    rbg = 512 if n % 512 == 0 else n
    c3 = cin + 3
    return pl.pallas_call(
        _gmat_body,
        grid=(n // rbg,),
        in_specs=[
            pl.BlockSpec((rbg, c3), lambda i: (i, 0)),
            pl.BlockSpec((rbg, 8), lambda i: (i, 0)),
            pl.BlockSpec((rbg, cin), lambda i: (i, 0)),
            pl.BlockSpec((c3, inner), lambda i: (0, 0)),
            pl.BlockSpec((8, inner), lambda i: (0, 0)),
            pl.BlockSpec((cin, inner), lambda i: (0, 0)),
        ],
        out_specs=[
            pl.BlockSpec((rbg, inner), lambda i: (i, 0)),
            pl.BlockSpec((rbg, inner), lambda i: (i, 0)),
            pl.BlockSpec((rbg, inner), lambda i: (i, 0)),
        ],
        out_shape=[
            jax.ShapeDtypeStruct((n, inner), jnp.float32),
            jax.ShapeDtypeStruct((n, inner), jnp.float32),
            jax.ShapeDtypeStruct((n, inner), jnp.float32),
        ],
    )(x_cat, pts_pad, feature, cwT, w3T, l1bT)


def _gather_rows(table, idx_flat):
    nrows = idx_flat.shape[0]
    d = table.shape[1]
    nw = 32
    per_w = nrows // nw
    ch = 128
    nch = per_w // ch
    mesh = plsc.VectorSubcoreMesh(core_axis_name="c", subcore_axis_name="s")

    @functools.partial(
        pl.kernel, mesh=mesh,
        out_type=jax.ShapeDtypeStruct((nrows, d), jnp.float32),
        scratch_types=[
            pltpu.VMEM((per_w,), jnp.int32),
            pltpu.VMEM((ch, d), jnp.float32),
            pltpu.VMEM((ch, d), jnp.float32),
            pltpu.SemaphoreType.DMA,
            pltpu.SemaphoreType.DMA,
            pltpu.SemaphoreType.DMA,
            pltpu.SemaphoreType.DMA,
        ],
    )
    def gk(table_hbm, idx_hbm, out_hbm, idx_all, rows0, rows1,
           sem0, sem1, osem0, osem1):
        wid = lax.axis_index("s") * 2 + lax.axis_index("c")
        base = wid * per_w
        rows_v = [rows0, rows1]
        sems = [sem0, sem1]
        osems = [osem0, osem1]
        pltpu.sync_copy(idx_hbm.at[pl.ds(base, per_w)], idx_all)

        def start(c, slot):
            pltpu.async_copy(
                table_hbm.at[idx_all.at[pl.ds(c * ch, ch)]],
                rows_v[slot], sems[slot])

        def handle(c, slot):
            pltpu.make_async_copy(
                table_hbm.at[idx_all.at[pl.ds(c * ch, ch)]],
                rows_v[slot], sems[slot]).wait()
            pltpu.async_copy(rows_v[slot],
                             out_hbm.at[pl.ds(base + c * ch, ch)],
                             osems[slot])

        def owait(c, slot):
            pltpu.make_async_copy(
                rows_v[slot], out_hbm.at[pl.ds(base + c * ch, ch)],
                osems[slot]).wait()

        start(0, 0)

        def body(c, carry):
            slot = lax.rem(c, 2)

            @pl.when(c >= 1)
            def _():
                jax.lax.switch(1 - slot, [lambda: owait(c - 1, 0),
                                          lambda: owait(c - 1, 1)])

            @pl.when(c + 1 < nch)
            def _():
                jax.lax.switch(1 - slot, [lambda: start(c + 1, 0),
                                          lambda: start(c + 1, 1)])

            jax.lax.switch(slot, [lambda: handle(c, 0), lambda: handle(c, 1)])
            return carry

        lax.fori_loop(0, nch, body, 0)
        owait(nch - 1, (nch - 1) % 2)

    return gk(table, idx_flat)


def _stats1_body(h_ref, p2_ref, ms_ref, out_ref, *, inner):
    i = pl.program_id(0)
    h = h_ref[...]
    p2 = p2_ref[...]
    hs = jnp.dot(ms_ref[...], h, preferred_element_type=jnp.float32)
    s1 = jnp.sum(h, axis=0, keepdims=True)
    s2 = jnp.sum(h * h, axis=0, keepdims=True)
    s3 = jnp.sum(p2 * hs, axis=0, keepdims=True)
    s4 = jnp.sum(p2, axis=0, keepdims=True)
    s5 = jnp.sum(p2 * p2, axis=0, keepdims=True)
    contrib = jnp.concatenate(
        [s1, s2, s3, s4, s5, jnp.zeros((3, inner), jnp.float32)], axis=0)

    @pl.when(i == 0)
    def _():
        out_ref[...] = jnp.zeros_like(out_ref)

    out_ref[...] += contrib


def _stats1(h, p2, msum, n, inner):
    pb = 128 if n % 128 == 0 else n
    return pl.pallas_call(
        functools.partial(_stats1_body, inner=inner),
        grid=(n // pb,),
        in_specs=[
            pl.BlockSpec((pb * _K, inner), lambda i: (i, 0)),
            pl.BlockSpec((pb, inner), lambda i: (i, 0)),
            pl.BlockSpec((pb, pb * _K), lambda i: (0, 0)),
        ],
        out_specs=pl.BlockSpec((8, inner), lambda i: (0, 0)),
        out_shape=jax.ShapeDtypeStruct((8, inner), jnp.float32),
    )(h, p2, msum)


def _passb_body(h_ref, p2_ref, a_ref, c_ref, y_ref, w_ref, *, pb, inner):
    a = a_ref[...]
    u = c_ref[...] - a * p2_ref[...]
    s = a * h_ref[:, 0, :] + u
    y = None
    wcols = []
    for k in range(_K):
        xk = a * h_ref[:, k, :] + u
        wk = jnp.sum(xk * s, axis=1, keepdims=True)
        wcols.append(wk)
        yk = xk * wk
        y = yk if y is None else y + yk
    w_ref[...] = jnp.concatenate(wcols, axis=1)
    y_ref[...] = y


def _passb(h3, p2, a1, c1, n, inner):
    pb = 128 if n % 128 == 0 else n
    return pl.pallas_call(
        functools.partial(_passb_body, pb=pb, inner=inner),
        grid=(n // pb,),
        in_specs=[
            pl.BlockSpec((pb, _K, inner), lambda i: (i, 0, 0)),
            pl.BlockSpec((pb, inner), lambda i: (i, 0)),
            pl.BlockSpec((1, inner), lambda i: (0, 0)),
            pl.BlockSpec((1, inner), lambda i: (0, 0)),
        ],
        out_specs=[
            pl.BlockSpec((pb, inner), lambda i: (i, 0)),
            pl.BlockSpec((pb, _K), lambda i: (i, 0)),
        ],
        out_shape=[
            jax.ShapeDtypeStruct((n, inner), jnp.float32),
            jax.ShapeDtypeStruct((n, _K), jnp.float32),
        ],
    )(h3, p2, a1, c1)


def _passc_body(h2_ref, wr_ref, ms_ref, z_ref, acc_ref, *, inner):
    i = pl.program_id(0)
    hw = h2_ref[...] * wr_ref[...]
    z = jnp.dot(ms_ref[...], hw, preferred_element_type=jnp.float32)
    z_ref[...] = z
    contrib = jnp.concatenate(
        [jnp.sum(z, axis=0, keepdims=True),
         jnp.sum(z * z, axis=0, keepdims=True),
         jnp.zeros((6, inner), jnp.float32)], axis=0)

    @pl.when(i == 0)
    def _():
        acc_ref[...] = jnp.zeros_like(acc_ref)

    acc_ref[...] += contrib


def _passc(h2, wr, msum, n, inner):
    pb = 128 if n % 128 == 0 else n
    return pl.pallas_call(
        functools.partial(_passc_body, inner=inner),
        grid=(n // pb,),
        in_specs=[
            pl.BlockSpec((pb * _K, inner), lambda i: (i, 0)),
            pl.BlockSpec((pb * _K, 1), lambda i: (i, 0)),
            pl.BlockSpec((pb, pb * _K), lambda i: (0, 0)),
        ],
        out_specs=[
            pl.BlockSpec((pb, inner), lambda i: (i, 0)),
            pl.BlockSpec((8, inner), lambda i: (0, 0)),
        ],
        out_shape=[
            jax.ShapeDtypeStruct((n, inner), jnp.float32),
            jax.ShapeDtypeStruct((8, inner), jnp.float32),
        ],
    )(h2, wr, msum)


def _passd1_body(z_ref, f3_ref, a2_ref, c2_ref, l1aT_ref, b1_ref,
                 t_ref, acc_ref, *, cin):
    i = pl.program_id(0)
    r = jnp.maximum(a2_ref[...] * z_ref[...] + c2_ref[...], 0.0)
    t = (jnp.dot(r, l1aT_ref[...], preferred_element_type=jnp.float32)
         + f3_ref[...] + b1_ref[...])
    t_ref[...] = t
    contrib = jnp.concatenate(
        [jnp.sum(t, axis=0, keepdims=True),
         jnp.sum(t * t, axis=0, keepdims=True),
         jnp.zeros((6, cin), jnp.float32)], axis=0)

    @pl.when(i == 0)
    def _():
        acc_ref[...] = jnp.zeros_like(acc_ref)

    acc_ref[...] += contrib


def _passd1(z, f3, a2, c2, l1aT, b1, n, cin, inner):
    rbg = 512 if n % 512 == 0 else n
    return pl.pallas_call(
        functools.partial(_passd1_body, cin=cin),
        grid=(n // rbg,),
        in_specs=[
            pl.BlockSpec((rbg, inner), lambda i: (i, 0)),
            pl.BlockSpec((rbg, cin), lambda i: (i, 0)),
            pl.BlockSpec((1, inner), lambda i: (0, 0)),
            pl.BlockSpec((1, inner), lambda i: (0, 0)),
            pl.BlockSpec((inner, cin), lambda i: (0, 0)),
            pl.BlockSpec((1, cin), lambda i: (0, 0)),
        ],
        out_specs=[
            pl.BlockSpec((rbg, cin), lambda i: (i, 0)),
            pl.BlockSpec((8, cin), lambda i: (0, 0)),
        ],
        out_shape=[
            jax.ShapeDtypeStruct((n, cin), jnp.float32),
            jax.ShapeDtypeStruct((8, cin), jnp.float32),
        ],
    )(z, f3, a2, c2, l1aT, b1)


def _passd2_body(t_ref, a3_ref, c3_ref, l2T_ref, b2_ref, o_ref):
    r = jnp.maximum(a3_ref[...] * t_ref[...] + c3_ref[...], 0.0)
    o_ref[...] = (jnp.dot(r, l2T_ref[...], preferred_element_type=jnp.float32)
                  + b2_ref[...])


def _passd2(t, a3, c3, l2T, b2, n, cin):
    rbg = 512 if n % 512 == 0 else n
    return pl.pallas_call(
        _passd2_body,
        grid=(n // rbg,),
        in_specs=[
            pl.BlockSpec((rbg, cin), lambda i: (i, 0)),
            pl.BlockSpec((1, cin), lambda i: (0, 0)),
            pl.BlockSpec((1, cin), lambda i: (0, 0)),
            pl.BlockSpec((cin, cin), lambda i: (0, 0)),
            pl.BlockSpec((1, cin), lambda i: (0, 0)),
        ],
        out_specs=pl.BlockSpec((rbg, cin), lambda i: (i, 0)),
        out_shape=jax.ShapeDtypeStruct((n, cin), jnp.float32),
    )(t, a3, c3, l2T, b2)


def kernel(coords, points, feature, conv_w, bn1_g, bn1_b, bn2_g, bn2_b,
           lin1_w, lin1_b, bn3_g, bn3_b, lin2_w, lin2_b):
    n, cin = feature.shape
    inner = conv_w.shape[0]
    f32 = jnp.float32
    b = coords[:, 3].astype(jnp.int32)

    ptsT = jnp.concatenate([points.T, jnp.zeros((5, n), f32)], axis=0)
    rb = 256 if n % 256 == 0 else n
    ct = 1024 if n % 1024 == 0 else n
    tr = _tile_ranges(b, n, rb, ct)
    nh = n // 2
    b2d = b[None, :]
    bcol = b[:, None]
    idx_lo = _knn(ptsT, b2d, bcol, tr, n, 0, nh)
    idx_hi = _knn(ptsT, b2d, bcol, tr, n, nh // rb, nh)

    x_cat = jnp.concatenate([feature, points], axis=1)
    pts_pad = jnp.concatenate([points, jnp.zeros((n, 5), f32)], axis=1)
    w3T = jnp.concatenate(
        [conv_w[:, cin:].T, jnp.zeros((5, inner), f32)], axis=0)
    g, p2, f3 = _gmat(x_cat, pts_pad, feature, conv_w.T, w3T,
                      lin1_w[:, inner:].T, n, cin, inner)

    pb = 128 if n % 128 == 0 else n
    msum = (jnp.arange(pb, dtype=jnp.int32)[:, None]
            == (jnp.arange(pb * _K, dtype=jnp.int32)[None, :] // _K)
            ).astype(f32)
    h_lo = _gather_rows(g, idx_lo.reshape(-1))
    h_hi = _gather_rows(g, idx_hi.reshape(-1))
    p2_lo, p2_hi = p2[:nh], p2[nh:]
    s = (_stats1(h_lo, p2_lo, msum, nh, inner)
         + _stats1(h_hi, p2_hi, msum, nh, inner))
    nk = jnp.float32(n * _K)
    mean1 = (s[0] - _K * s[3]) / nk
    ex2 = (s[1] - 2.0 * s[2] + _K * s[4]) / nk
    var1 = ex2 - mean1 * mean1
    a1 = bn1_g / jnp.sqrt(var1 + _EPS)
    c1 = bn1_b - a1 * mean1

    y_lo, w_lo = _passb(h_lo.reshape(nh, _K, inner), p2_lo,
                        a1[None], c1[None], nh, inner)
    y_hi, w_hi = _passb(h_hi.reshape(nh, _K, inner), p2_hi,
                        a1[None], c1[None], nh, inner)
    y = jnp.concatenate([y_lo, y_hi], axis=0)

    h2_lo = _gather_rows(y, idx_lo.reshape(-1))
    h2_hi = _gather_rows(y, idx_hi.reshape(-1))
    z_lo, acc2_lo = _passc(h2_lo, w_lo.reshape(nh * _K, 1), msum, nh, inner)
    z_hi, acc2_hi = _passc(h2_hi, w_hi.reshape(nh * _K, 1), msum, nh, inner)
    z = jnp.concatenate([z_lo, z_hi], axis=0)
    acc2 = acc2_lo + acc2_hi
    mean2 = acc2[0] / n
    var2 = acc2[1] / n - mean2 * mean2
    a2 = bn2_g / jnp.sqrt(var2 + _EPS)
    c2 = bn2_b - a2 * mean2

    t, acc3 = _passd1(z, f3, a2[None], c2[None], lin1_w[:, :inner].T,
                      lin1_b[None], n, cin, inner)
    mean3 = acc3[0] / n
    var3 = acc3[1] / n - mean3 * mean3
    a3 = bn3_g / jnp.sqrt(var3 + _EPS)
    c3 = bn3_b - a3 * mean3

    return _passd2(t, a3[None], c3[None], lin2_w.T, lin2_b[None], n, cin)


# speedup vs baseline: 1.3450x; 1.0232x over previous
import functools

import jax
import jax.numpy as jnp
from jax import lax
from jax.experimental import pallas as pl
from jax.experimental.pallas import tpu as pltpu
from jax.experimental.pallas import tpu_sc as plsc

_EPS = 1e-5
_K = 16
_BIG = 2**30


def _knn_body(tr_ref, ptsT_ref, b2d_ref, bcol_ref, idx_ref, d2_ref,
              *, rb, ct, n, i0):
    i = pl.program_id(0) + i0
    tlo = tr_ref[2 * i]
    thi = tr_ref[2 * i + 1]
    prow = ptsT_ref[:, pl.ds(i * rb, rb)]
    sqrow = lax.dot_general(
        prow * prow, jnp.ones((8, 1), jnp.float32),
        (((0,), (0,)), ((), ())), preferred_element_type=jnp.float32)
    bcol = bcol_ref[...]
    m0 = jnp.full((rb, 1), jnp.inf, jnp.float32)
    a0 = jnp.full((rb, 1), _BIG, jnp.int32)
    iota_c = lax.broadcasted_iota(jnp.int32, (rb, ct), 1)
    lane_k = lax.broadcasted_iota(jnp.int32, (rb, _K), 1)

    def merge(carry, mt, cc):
        m, am = carry
        am2 = jnp.where(mt < m, cc,
                        jnp.where(mt == m, jnp.minimum(am, cc), am))
        return jnp.minimum(m, mt), am2

    def init_body(t, carry):
        off = pl.multiple_of(t * ct, ct)
        ptile = ptsT_ref[:, pl.ds(off, ct)]
        pp = lax.dot_general(prow, ptile, (((0,), (0,)), ((), ())),
                             preferred_element_type=jnp.float32)
        sqcol = jnp.sum(ptile * ptile, axis=0, keepdims=True)
        d2 = sqrow + sqcol - 2.0 * pp
        same = bcol == b2d_ref[:, pl.ds(off, ct)]
        tile = jnp.where(same, d2, jnp.inf)
        d2_ref[:, pl.ds(off, ct)] = tile
        it = iota_c + t * ct
        mt = jnp.min(tile, axis=1, keepdims=True)
        cc = jnp.min(jnp.where(tile == mt, it, _BIG), axis=1, keepdims=True)
        return merge(carry, mt, cc)

    m, am = lax.fori_loop(tlo, thi, init_body, (m0, a0))
    acc = jnp.where(lane_k == 0, am, 0)

    def kstep(k, carry):
        acc, aprev = carry

        def sbody(t, c):
            off = pl.multiple_of(t * ct, ct)
            tile = d2_ref[:, pl.ds(off, ct)]
            it = iota_c + t * ct
            tile = jnp.where(it == aprev, jnp.inf, tile)
            d2_ref[:, pl.ds(off, ct)] = tile
            mt = jnp.min(tile, axis=1, keepdims=True)
            cc = jnp.min(jnp.where(tile == mt, it, _BIG), axis=1, keepdims=True)
            return merge(c, mt, cc)

        m, am = lax.fori_loop(tlo, thi, sbody, (m0, a0))
        acc = jnp.where(lane_k == k, am, acc)
        return acc, am

    acc, _ = lax.fori_loop(1, _K, kstep, (acc, am))
    idx_ref[...] = acc


def _knn(ptsT, b2d, bcol, tr, n, i0, nout):
    rb = 256 if n % 256 == 0 else n
    ct = 2048 if n % 2048 == 0 else n
    grid_spec = pltpu.PrefetchScalarGridSpec(
        num_scalar_prefetch=1,
        grid=(nout // rb,),
        in_specs=[
            pl.BlockSpec((8, n), lambda i, *_: (0, 0)),
            pl.BlockSpec((1, n), lambda i, *_: (0, 0)),
            pl.BlockSpec((rb, 1), lambda i, *_: (i + i0, 0)),
        ],
        out_specs=pl.BlockSpec((rb, _K), lambda i, *_: (i, 0)),
        scratch_shapes=[pltpu.VMEM((rb, n), jnp.float32)],
    )
    return pl.pallas_call(
        functools.partial(_knn_body, rb=rb, ct=ct, n=n, i0=i0),
        grid_spec=grid_spec,
        out_shape=jax.ShapeDtypeStruct((nout, _K), jnp.int32),
    )(tr, ptsT, b2d, bcol)


def _tile_ranges(b, n, rb, ct):
    nblk = n // rb
    ntiles = n // ct
    bb = b.reshape(nblk, rb)
    bcast = b[None, :]
    lo_col = jnp.sum((bcast < bb[:, 0][:, None]).astype(jnp.int32), axis=1)
    hi_col = jnp.sum((bcast <= bb[:, -1][:, None]).astype(jnp.int32), axis=1)
    vals = jnp.arange(8, dtype=jnp.int32)
    hist = jnp.sum((bcast == vals[:, None]).astype(jnp.int32), axis=1)
    minsz = jnp.min(jnp.where(hist > 0, hist, n))
    tlo = jnp.where(minsz < _K, 0, lo_col // ct)
    thi = jnp.where(minsz < _K, ntiles, (hi_col + ct - 1) // ct)
    return jnp.stack([tlo, thi], axis=1).reshape(-1).astype(jnp.int32)


def _gmat_body(x_ref, pts_ref, feat_ref, cwT_ref, w3T_ref, l1bT_ref,
               g_ref, p2_ref, f3_ref):
    g_ref[...] = jnp.dot(x_ref[...], cwT_ref[...],
                         preferred_element_type=jnp.float32)
    p2_ref[...] = jnp.dot(pts_ref[...], w3T_ref[...],
                          preferred_element_type=jnp.float32)
    f3_ref[...] = jnp.dot(feat_ref[...], l1bT_ref[...],
                          preferred_element_type=jnp.float32)


def _gmat(x_cat, pts_pad, feature, cwT, w3T, l1bT, n, cin, inner):
    rbg = 512 if n % 512 == 0 else n
    c3 = cin + 3
    return pl.pallas_call(
        _gmat_body,
        grid=(n // rbg,),
        in_specs=[
            pl.BlockSpec((rbg, c3), lambda i: (i, 0)),
            pl.BlockSpec((rbg, 8), lambda i: (i, 0)),
            pl.BlockSpec((rbg, cin), lambda i: (i, 0)),
            pl.BlockSpec((c3, inner), lambda i: (0, 0)),
            pl.BlockSpec((8, inner), lambda i: (0, 0)),
            pl.BlockSpec((cin, inner), lambda i: (0, 0)),
        ],
        out_specs=[
            pl.BlockSpec((rbg, inner), lambda i: (i, 0)),
            pl.BlockSpec((rbg, inner), lambda i: (i, 0)),
            pl.BlockSpec((rbg, inner), lambda i: (i, 0)),
        ],
        out_shape=[
            jax.ShapeDtypeStruct((n, inner), jnp.float32),
            jax.ShapeDtypeStruct((n, inner), jnp.float32),
            jax.ShapeDtypeStruct((n, inner), jnp.float32),
        ],
    )(x_cat, pts_pad, feature, cwT, w3T, l1bT)


def _gather_rows(table, idx_flat):
    nrows = idx_flat.shape[0]
    d = table.shape[1]
    nw = 32
    per_w = nrows // nw
    ch = 128
    nch = per_w // ch
    mesh = plsc.VectorSubcoreMesh(core_axis_name="c", subcore_axis_name="s")

    @functools.partial(
        pl.kernel, mesh=mesh,
        out_type=jax.ShapeDtypeStruct((nrows, d), jnp.float32),
        scratch_types=[
            pltpu.VMEM((per_w,), jnp.int32),
            pltpu.VMEM((ch, d), jnp.float32),
            pltpu.VMEM((ch, d), jnp.float32),
            pltpu.SemaphoreType.DMA,
            pltpu.SemaphoreType.DMA,
            pltpu.SemaphoreType.DMA,
            pltpu.SemaphoreType.DMA,
        ],
    )
    def gk(table_hbm, idx_hbm, out_hbm, idx_all, rows0, rows1,
           sem0, sem1, osem0, osem1):
        wid = lax.axis_index("s") * 2 + lax.axis_index("c")
        base = wid * per_w
        rows_v = [rows0, rows1]
        sems = [sem0, sem1]
        osems = [osem0, osem1]
        pltpu.sync_copy(idx_hbm.at[pl.ds(base, per_w)], idx_all)

        def start(c, slot):
            pltpu.async_copy(
                table_hbm.at[idx_all.at[pl.ds(c * ch, ch)]],
                rows_v[slot], sems[slot])

        def handle(c, slot):
            pltpu.make_async_copy(
                table_hbm.at[idx_all.at[pl.ds(c * ch, ch)]],
                rows_v[slot], sems[slot]).wait()
            pltpu.async_copy(rows_v[slot],
                             out_hbm.at[pl.ds(base + c * ch, ch)],
                             osems[slot])

        def owait(c, slot):
            pltpu.make_async_copy(
                rows_v[slot], out_hbm.at[pl.ds(base + c * ch, ch)],
                osems[slot]).wait()

        start(0, 0)

        def body(c, carry):
            slot = lax.rem(c, 2)

            @pl.when(c >= 1)
            def _():
                jax.lax.switch(1 - slot, [lambda: owait(c - 1, 0),
                                          lambda: owait(c - 1, 1)])

            @pl.when(c + 1 < nch)
            def _():
                jax.lax.switch(1 - slot, [lambda: start(c + 1, 0),
                                          lambda: start(c + 1, 1)])

            jax.lax.switch(slot, [lambda: handle(c, 0), lambda: handle(c, 1)])
            return carry

        lax.fori_loop(0, nch, body, 0)
        owait(nch - 1, (nch - 1) % 2)

    return gk(table, idx_flat)


def _stats1_body(h_ref, p2_ref, ms_ref, out_ref, *, inner):
    i = pl.program_id(0)
    h = h_ref[...]
    p2 = p2_ref[...]
    hs = jnp.dot(ms_ref[...], h, preferred_element_type=jnp.float32)
    s1 = jnp.sum(h, axis=0, keepdims=True)
    s2 = jnp.sum(h * h, axis=0, keepdims=True)
    s3 = jnp.sum(p2 * hs, axis=0, keepdims=True)
    s4 = jnp.sum(p2, axis=0, keepdims=True)
    s5 = jnp.sum(p2 * p2, axis=0, keepdims=True)
    contrib = jnp.concatenate(
        [s1, s2, s3, s4, s5, jnp.zeros((3, inner), jnp.float32)], axis=0)

    @pl.when(i == 0)
    def _():
        out_ref[...] = jnp.zeros_like(out_ref)

    out_ref[...] += contrib


def _stats1(h, p2, msum, n, inner):
    pb = 128 if n % 128 == 0 else n
    return pl.pallas_call(
        functools.partial(_stats1_body, inner=inner),
        grid=(n // pb,),
        in_specs=[
            pl.BlockSpec((pb * _K, inner), lambda i: (i, 0)),
            pl.BlockSpec((pb, inner), lambda i: (i, 0)),
            pl.BlockSpec((pb, pb * _K), lambda i: (0, 0)),
        ],
        out_specs=pl.BlockSpec((8, inner), lambda i: (0, 0)),
        out_shape=jax.ShapeDtypeStruct((8, inner), jnp.float32),
    )(h, p2, msum)


def _passb_body(h_ref, p2_ref, a_ref, c_ref, y_ref, w_ref, *, pb, inner):
    a = a_ref[...]
    u = c_ref[...] - a * p2_ref[...]
    s = a * h_ref[:, 0, :] + u
    y = None
    wcols = []
    for k in range(_K):
        xk = a * h_ref[:, k, :] + u
        wk = jnp.sum(xk * s, axis=1, keepdims=True)
        wcols.append(wk)
        yk = xk * wk
        y = yk if y is None else y + yk
    w_ref[...] = jnp.concatenate(wcols, axis=1)
    y_ref[...] = y


def _passb(h3, p2, a1, c1, n, inner):
    pb = 128 if n % 128 == 0 else n
    return pl.pallas_call(
        functools.partial(_passb_body, pb=pb, inner=inner),
        grid=(n // pb,),
        in_specs=[
            pl.BlockSpec((pb, _K, inner), lambda i: (i, 0, 0)),
            pl.BlockSpec((pb, inner), lambda i: (i, 0)),
            pl.BlockSpec((1, inner), lambda i: (0, 0)),
            pl.BlockSpec((1, inner), lambda i: (0, 0)),
        ],
        out_specs=[
            pl.BlockSpec((pb, inner), lambda i: (i, 0)),
            pl.BlockSpec((pb, _K), lambda i: (i, 0)),
        ],
        out_shape=[
            jax.ShapeDtypeStruct((n, inner), jnp.float32),
            jax.ShapeDtypeStruct((n, _K), jnp.float32),
        ],
    )(h3, p2, a1, c1)


def _passc_body(h2_ref, wr_ref, ms_ref, z_ref, acc_ref, *, inner):
    i = pl.program_id(0)
    hw = h2_ref[...] * wr_ref[...]
    z = jnp.dot(ms_ref[...], hw, preferred_element_type=jnp.float32)
    z_ref[...] = z
    contrib = jnp.concatenate(
        [jnp.sum(z, axis=0, keepdims=True),
         jnp.sum(z * z, axis=0, keepdims=True),
         jnp.zeros((6, inner), jnp.float32)], axis=0)

    @pl.when(i == 0)
    def _():
        acc_ref[...] = jnp.zeros_like(acc_ref)

    acc_ref[...] += contrib


def _passc(h2, wr, msum, n, inner):
    pb = 128 if n % 128 == 0 else n
    return pl.pallas_call(
        functools.partial(_passc_body, inner=inner),
        grid=(n // pb,),
        in_specs=[
            pl.BlockSpec((pb * _K, inner), lambda i: (i, 0)),
            pl.BlockSpec((pb * _K, 1), lambda i: (i, 0)),
            pl.BlockSpec((pb, pb * _K), lambda i: (0, 0)),
        ],
        out_specs=[
            pl.BlockSpec((pb, inner), lambda i: (i, 0)),
            pl.BlockSpec((8, inner), lambda i: (0, 0)),
        ],
        out_shape=[
            jax.ShapeDtypeStruct((n, inner), jnp.float32),
            jax.ShapeDtypeStruct((8, inner), jnp.float32),
        ],
    )(h2, wr, msum)


def _passd1_body(z_ref, f3_ref, a2_ref, c2_ref, l1aT_ref, b1_ref,
                 t_ref, acc_ref, *, cin):
    i = pl.program_id(0)
    r = jnp.maximum(a2_ref[...] * z_ref[...] + c2_ref[...], 0.0)
    t = (jnp.dot(r, l1aT_ref[...], preferred_element_type=jnp.float32)
         + f3_ref[...] + b1_ref[...])
    t_ref[...] = t
    contrib = jnp.concatenate(
        [jnp.sum(t, axis=0, keepdims=True),
         jnp.sum(t * t, axis=0, keepdims=True),
         jnp.zeros((6, cin), jnp.float32)], axis=0)

    @pl.when(i == 0)
    def _():
        acc_ref[...] = jnp.zeros_like(acc_ref)

    acc_ref[...] += contrib


def _passd1(z, f3, a2, c2, l1aT, b1, n, cin, inner):
    rbg = 512 if n % 512 == 0 else n
    return pl.pallas_call(
        functools.partial(_passd1_body, cin=cin),
        grid=(n // rbg,),
        in_specs=[
            pl.BlockSpec((rbg, inner), lambda i: (i, 0)),
            pl.BlockSpec((rbg, cin), lambda i: (i, 0)),
            pl.BlockSpec((1, inner), lambda i: (0, 0)),
            pl.BlockSpec((1, inner), lambda i: (0, 0)),
            pl.BlockSpec((inner, cin), lambda i: (0, 0)),
            pl.BlockSpec((1, cin), lambda i: (0, 0)),
        ],
        out_specs=[
            pl.BlockSpec((rbg, cin), lambda i: (i, 0)),
            pl.BlockSpec((8, cin), lambda i: (0, 0)),
        ],
        out_shape=[
            jax.ShapeDtypeStruct((n, cin), jnp.float32),
            jax.ShapeDtypeStruct((8, cin), jnp.float32),
        ],
    )(z, f3, a2, c2, l1aT, b1)


def _passd2_body(t_ref, a3_ref, c3_ref, l2T_ref, b2_ref, o_ref):
    r = jnp.maximum(a3_ref[...] * t_ref[...] + c3_ref[...], 0.0)
    o_ref[...] = (jnp.dot(r, l2T_ref[...], preferred_element_type=jnp.float32)
                  + b2_ref[...])


def _passd2(t, a3, c3, l2T, b2, n, cin):
    rbg = 512 if n % 512 == 0 else n
    return pl.pallas_call(
        _passd2_body,
        grid=(n // rbg,),
        in_specs=[
            pl.BlockSpec((rbg, cin), lambda i: (i, 0)),
            pl.BlockSpec((1, cin), lambda i: (0, 0)),
            pl.BlockSpec((1, cin), lambda i: (0, 0)),
            pl.BlockSpec((cin, cin), lambda i: (0, 0)),
            pl.BlockSpec((1, cin), lambda i: (0, 0)),
        ],
        out_specs=pl.BlockSpec((rbg, cin), lambda i: (i, 0)),
        out_shape=jax.ShapeDtypeStruct((n, cin), jnp.float32),
    )(t, a3, c3, l2T, b2)


def kernel(coords, points, feature, conv_w, bn1_g, bn1_b, bn2_g, bn2_b,
           lin1_w, lin1_b, bn3_g, bn3_b, lin2_w, lin2_b):
    n, cin = feature.shape
    inner = conv_w.shape[0]
    f32 = jnp.float32
    b = coords[:, 3].astype(jnp.int32)

    ptsT = jnp.concatenate([points.T, jnp.zeros((5, n), f32)], axis=0)
    rb = 256 if n % 256 == 0 else n
    ct = 2048 if n % 2048 == 0 else n
    tr = _tile_ranges(b, n, rb, ct)
    nh = n // 2
    b2d = b[None, :]
    bcol = b[:, None]
    idx_lo = _knn(ptsT, b2d, bcol, tr, n, 0, nh)
    idx_hi = _knn(ptsT, b2d, bcol, tr, n, nh // rb, nh)

    x_cat = jnp.concatenate([feature, points], axis=1)
    pts_pad = jnp.concatenate([points, jnp.zeros((n, 5), f32)], axis=1)
    w3T = jnp.concatenate(
        [conv_w[:, cin:].T, jnp.zeros((5, inner), f32)], axis=0)
    g, p2, f3 = _gmat(x_cat, pts_pad, feature, conv_w.T, w3T,
                      lin1_w[:, inner:].T, n, cin, inner)

    pb = 128 if n % 128 == 0 else n
    msum = (jnp.arange(pb, dtype=jnp.int32)[:, None]
            == (jnp.arange(pb * _K, dtype=jnp.int32)[None, :] // _K)
            ).astype(f32)
    h_lo = _gather_rows(g, idx_lo.reshape(-1))
    h_hi = _gather_rows(g, idx_hi.reshape(-1))
    p2_lo, p2_hi = p2[:nh], p2[nh:]
    s = (_stats1(h_lo, p2_lo, msum, nh, inner)
         + _stats1(h_hi, p2_hi, msum, nh, inner))
    nk = jnp.float32(n * _K)
    mean1 = (s[0] - _K * s[3]) / nk
    ex2 = (s[1] - 2.0 * s[2] + _K * s[4]) / nk
    var1 = ex2 - mean1 * mean1
    a1 = bn1_g / jnp.sqrt(var1 + _EPS)
    c1 = bn1_b - a1 * mean1

    y_lo, w_lo = _passb(h_lo.reshape(nh, _K, inner), p2_lo,
                        a1[None], c1[None], nh, inner)
    y_hi, w_hi = _passb(h_hi.reshape(nh, _K, inner), p2_hi,
                        a1[None], c1[None], nh, inner)
    y = jnp.concatenate([y_lo, y_hi], axis=0)

    h2_lo = _gather_rows(y, idx_lo.reshape(-1))
    h2_hi = _gather_rows(y, idx_hi.reshape(-1))
    z_lo, acc2_lo = _passc(h2_lo, w_lo.reshape(nh * _K, 1), msum, nh, inner)
    z_hi, acc2_hi = _passc(h2_hi, w_hi.reshape(nh * _K, 1), msum, nh, inner)
    z = jnp.concatenate([z_lo, z_hi], axis=0)
    acc2 = acc2_lo + acc2_hi
    mean2 = acc2[0] / n
    var2 = acc2[1] / n - mean2 * mean2
    a2 = bn2_g / jnp.sqrt(var2 + _EPS)
    c2 = bn2_b - a2 * mean2

    t, acc3 = _passd1(z, f3, a2[None], c2[None], lin1_w[:, :inner].T,
                      lin1_b[None], n, cin, inner)
    mean3 = acc3[0] / n
    var3 = acc3[1] / n - mean3 * mean3
    a3 = bn3_g / jnp.sqrt(var3 + _EPS)
    c3 = bn3_b - a3 * mean3

    return _passd2(t, a3[None], c3[None], lin2_w.T, lin2_b[None], n, cin)


# speedup vs baseline: 1.3508x; 1.0043x over previous
import functools

import jax
import jax.numpy as jnp
from jax import lax
from jax.experimental import pallas as pl
from jax.experimental.pallas import tpu as pltpu
from jax.experimental.pallas import tpu_sc as plsc

_EPS = 1e-5
_K = 16
_BIG = 2**30


def _knn_body(tr_ref, ptsT_ref, b2d_ref, bcol_ref, idx_ref, d2_ref,
              *, rb, ct, n, i0):
    i = pl.program_id(0) + i0
    tlo = tr_ref[2 * i]
    thi = tr_ref[2 * i + 1]
    prow = ptsT_ref[:, pl.ds(i * rb, rb)]
    sqrow = lax.dot_general(
        prow * prow, jnp.ones((8, 1), jnp.float32),
        (((0,), (0,)), ((), ())), preferred_element_type=jnp.float32)
    bcol = bcol_ref[...]
    m0 = jnp.full((rb, 1), jnp.inf, jnp.float32)
    a0 = jnp.full((rb, 1), _BIG, jnp.int32)
    iota_c = lax.broadcasted_iota(jnp.int32, (rb, ct), 1)
    lane_k = lax.broadcasted_iota(jnp.int32, (rb, _K), 1)

    def lexlt(v1, i1, v2, i2):
        return (v1 < v2) | ((v1 == v2) & (i1 < i2))

    def lexmin(v1, i1, v2, i2):
        p = lexlt(v1, i1, v2, i2)
        return jnp.where(p, v1, v2), jnp.where(p, i1, i2)

    def merge2(c, mt1, cc1, mt2, cc2):
        M1, A1, M2, A2 = c
        p = lexlt(M1, A1, mt1, cc1)
        f_v = jnp.where(p, M1, mt1)
        f_i = jnp.where(p, A1, cc1)
        sa_v, sa_i = lexmin(M2, A2, mt1, cc1)
        sb_v, sb_i = lexmin(M1, A1, mt2, cc2)
        s_v = jnp.where(p, sa_v, sb_v)
        s_i = jnp.where(p, sa_i, sb_i)
        return f_v, f_i, s_v, s_i

    def top2_of_tile(c, tile, it):
        mt1 = jnp.min(tile, axis=1, keepdims=True)
        cc1 = jnp.min(jnp.where(tile == mt1, it, _BIG), axis=1, keepdims=True)
        tile2 = jnp.where(it == cc1, jnp.inf, tile)
        mt2 = jnp.min(tile2, axis=1, keepdims=True)
        cc2 = jnp.min(jnp.where(tile2 == mt2, it, _BIG), axis=1, keepdims=True)
        return merge2(c, mt1, cc1, mt2, cc2)

    def init_body(t, carry):
        off = pl.multiple_of(t * ct, ct)
        ptile = ptsT_ref[:, pl.ds(off, ct)]
        pp = lax.dot_general(prow, ptile, (((0,), (0,)), ((), ())),
                             preferred_element_type=jnp.float32)
        sqcol = jnp.sum(ptile * ptile, axis=0, keepdims=True)
        d2 = sqrow + sqcol - 2.0 * pp
        same = bcol == b2d_ref[:, pl.ds(off, ct)]
        tile = jnp.where(same, d2, jnp.inf)
        d2_ref[:, pl.ds(off, ct)] = tile
        return top2_of_tile(carry, tile, iota_c + t * ct)

    c0 = (m0, a0, m0, a0)
    _, a1g, _, a2g = lax.fori_loop(tlo, thi, init_body, c0)
    acc = jnp.where(lane_k == 0, a1g, 0)
    acc = jnp.where(lane_k == 1, a2g, acc)

    def round_body(r, carry):
        acc, p1, p2 = carry

        def sbody(t, c):
            off = pl.multiple_of(t * ct, ct)
            tile = d2_ref[:, pl.ds(off, ct)]
            it = iota_c + t * ct
            tile = jnp.where((it == p1) | (it == p2), jnp.inf, tile)
            d2_ref[:, pl.ds(off, ct)] = tile
            return top2_of_tile(c, tile, it)

        _, b1, _, b2 = lax.fori_loop(tlo, thi, sbody, c0)
        acc = jnp.where(lane_k == 2 * r, b1, acc)
        acc = jnp.where(lane_k == 2 * r + 1, b2, acc)
        return acc, b1, b2

    acc, _, _ = lax.fori_loop(1, _K // 2, round_body, (acc, a1g, a2g))
    idx_ref[...] = acc


def _knn(ptsT, b2d, bcol, tr, n, i0, nout):
    rb = 256 if n % 256 == 0 else n
    ct = 2048 if n % 2048 == 0 else n
    grid_spec = pltpu.PrefetchScalarGridSpec(
        num_scalar_prefetch=1,
        grid=(nout // rb,),
        in_specs=[
            pl.BlockSpec((8, n), lambda i, *_: (0, 0)),
            pl.BlockSpec((1, n), lambda i, *_: (0, 0)),
            pl.BlockSpec((rb, 1), lambda i, *_: (i + i0, 0)),
        ],
        out_specs=pl.BlockSpec((rb, _K), lambda i, *_: (i, 0)),
        scratch_shapes=[pltpu.VMEM((rb, n), jnp.float32)],
    )
    return pl.pallas_call(
        functools.partial(_knn_body, rb=rb, ct=ct, n=n, i0=i0),
        grid_spec=grid_spec,
        out_shape=jax.ShapeDtypeStruct((nout, _K), jnp.int32),
    )(tr, ptsT, b2d, bcol)


def _tile_ranges(b, n, rb, ct):
    nblk = n // rb
    ntiles = n // ct
    bb = b.reshape(nblk, rb)
    bcast = b[None, :]
    lo_col = jnp.sum((bcast < bb[:, 0][:, None]).astype(jnp.int32), axis=1)
    hi_col = jnp.sum((bcast <= bb[:, -1][:, None]).astype(jnp.int32), axis=1)
    vals = jnp.arange(8, dtype=jnp.int32)
    hist = jnp.sum((bcast == vals[:, None]).astype(jnp.int32), axis=1)
    minsz = jnp.min(jnp.where(hist > 0, hist, n))
    tlo = jnp.where(minsz < _K, 0, lo_col // ct)
    thi = jnp.where(minsz < _K, ntiles, (hi_col + ct - 1) // ct)
    return jnp.stack([tlo, thi], axis=1).reshape(-1).astype(jnp.int32)


def _gmat_body(x_ref, pts_ref, feat_ref, cwT_ref, w3T_ref, l1bT_ref,
               g_ref, p2_ref, f3_ref):
    g_ref[...] = jnp.dot(x_ref[...], cwT_ref[...],
                         preferred_element_type=jnp.float32)
    p2_ref[...] = jnp.dot(pts_ref[...], w3T_ref[...],
                          preferred_element_type=jnp.float32)
    f3_ref[...] = jnp.dot(feat_ref[...], l1bT_ref[...],
                          preferred_element_type=jnp.float32)


def _gmat(x_cat, pts_pad, feature, cwT, w3T, l1bT, n, cin, inner):
    rbg = 512 if n % 512 == 0 else n
    c3 = cin + 3
    return pl.pallas_call(
        _gmat_body,
        grid=(n // rbg,),
        in_specs=[
            pl.BlockSpec((rbg, c3), lambda i: (i, 0)),
            pl.BlockSpec((rbg, 8), lambda i: (i, 0)),
            pl.BlockSpec((rbg, cin), lambda i: (i, 0)),
            pl.BlockSpec((c3, inner), lambda i: (0, 0)),
            pl.BlockSpec((8, inner), lambda i: (0, 0)),
            pl.BlockSpec((cin, inner), lambda i: (0, 0)),
        ],
        out_specs=[
            pl.BlockSpec((rbg, inner), lambda i: (i, 0)),
            pl.BlockSpec((rbg, inner), lambda i: (i, 0)),
            pl.BlockSpec((rbg, inner), lambda i: (i, 0)),
        ],
        out_shape=[
            jax.ShapeDtypeStruct((n, inner), jnp.float32),
            jax.ShapeDtypeStruct((n, inner), jnp.float32),
            jax.ShapeDtypeStruct((n, inner), jnp.float32),
        ],
    )(x_cat, pts_pad, feature, cwT, w3T, l1bT)


def _gather_rows(table, idx_flat):
    nrows = idx_flat.shape[0]
    d = table.shape[1]
    nw = 32
    per_w = nrows // nw
    ch = 128
    nch = per_w // ch
    mesh = plsc.VectorSubcoreMesh(core_axis_name="c", subcore_axis_name="s")

    @functools.partial(
        pl.kernel, mesh=mesh,
        out_type=jax.ShapeDtypeStruct((nrows, d), jnp.float32),
        scratch_types=[
            pltpu.VMEM((per_w,), jnp.int32),
            pltpu.VMEM((ch, d), jnp.float32),
            pltpu.VMEM((ch, d), jnp.float32),
            pltpu.SemaphoreType.DMA,
            pltpu.SemaphoreType.DMA,
            pltpu.SemaphoreType.DMA,
            pltpu.SemaphoreType.DMA,
        ],
    )
    def gk(table_hbm, idx_hbm, out_hbm, idx_all, rows0, rows1,
           sem0, sem1, osem0, osem1):
        wid = lax.axis_index("s") * 2 + lax.axis_index("c")
        base = wid * per_w
        rows_v = [rows0, rows1]
        sems = [sem0, sem1]
        osems = [osem0, osem1]
        pltpu.sync_copy(idx_hbm.at[pl.ds(base, per_w)], idx_all)

        def start(c, slot):
            pltpu.async_copy(
                table_hbm.at[idx_all.at[pl.ds(c * ch, ch)]],
                rows_v[slot], sems[slot])

        def handle(c, slot):
            pltpu.make_async_copy(
                table_hbm.at[idx_all.at[pl.ds(c * ch, ch)]],
                rows_v[slot], sems[slot]).wait()
            pltpu.async_copy(rows_v[slot],
                             out_hbm.at[pl.ds(base + c * ch, ch)],
                             osems[slot])

        def owait(c, slot):
            pltpu.make_async_copy(
                rows_v[slot], out_hbm.at[pl.ds(base + c * ch, ch)],
                osems[slot]).wait()

        start(0, 0)

        def body(c, carry):
            slot = lax.rem(c, 2)

            @pl.when(c >= 1)
            def _():
                jax.lax.switch(1 - slot, [lambda: owait(c - 1, 0),
                                          lambda: owait(c - 1, 1)])

            @pl.when(c + 1 < nch)
            def _():
                jax.lax.switch(1 - slot, [lambda: start(c + 1, 0),
                                          lambda: start(c + 1, 1)])

            jax.lax.switch(slot, [lambda: handle(c, 0), lambda: handle(c, 1)])
            return carry

        lax.fori_loop(0, nch, body, 0)
        owait(nch - 1, (nch - 1) % 2)

    return gk(table, idx_flat)


def _stats1_body(h_ref, p2_ref, ms_ref, out_ref, *, inner):
    i = pl.program_id(0)
    h = h_ref[...]
    p2 = p2_ref[...]
    hs = jnp.dot(ms_ref[...], h, preferred_element_type=jnp.float32)
    s1 = jnp.sum(h, axis=0, keepdims=True)
    s2 = jnp.sum(h * h, axis=0, keepdims=True)
    s3 = jnp.sum(p2 * hs, axis=0, keepdims=True)
    s4 = jnp.sum(p2, axis=0, keepdims=True)
    s5 = jnp.sum(p2 * p2, axis=0, keepdims=True)
    contrib = jnp.concatenate(
        [s1, s2, s3, s4, s5, jnp.zeros((3, inner), jnp.float32)], axis=0)

    @pl.when(i == 0)
    def _():
        out_ref[...] = jnp.zeros_like(out_ref)

    out_ref[...] += contrib


def _stats1(h, p2, msum, n, inner):
    pb = 128 if n % 128 == 0 else n
    return pl.pallas_call(
        functools.partial(_stats1_body, inner=inner),
        grid=(n // pb,),
        in_specs=[
            pl.BlockSpec((pb * _K, inner), lambda i: (i, 0)),
            pl.BlockSpec((pb, inner), lambda i: (i, 0)),
            pl.BlockSpec((pb, pb * _K), lambda i: (0, 0)),
        ],
        out_specs=pl.BlockSpec((8, inner), lambda i: (0, 0)),
        out_shape=jax.ShapeDtypeStruct((8, inner), jnp.float32),
    )(h, p2, msum)


def _passb_body(h_ref, p2_ref, a_ref, c_ref, y_ref, w_ref, *, pb, inner):
    a = a_ref[...]
    u = c_ref[...] - a * p2_ref[...]
    s = a * h_ref[:, 0, :] + u
    y = None
    wcols = []
    for k in range(_K):
        xk = a * h_ref[:, k, :] + u
        wk = jnp.sum(xk * s, axis=1, keepdims=True)
        wcols.append(wk)
        yk = xk * wk
        y = yk if y is None else y + yk
    w_ref[...] = jnp.concatenate(wcols, axis=1)
    y_ref[...] = y


def _passb(h3, p2, a1, c1, n, inner):
    pb = 128 if n % 128 == 0 else n
    return pl.pallas_call(
        functools.partial(_passb_body, pb=pb, inner=inner),
        grid=(n // pb,),
        in_specs=[
            pl.BlockSpec((pb, _K, inner), lambda i: (i, 0, 0)),
            pl.BlockSpec((pb, inner), lambda i: (i, 0)),
            pl.BlockSpec((1, inner), lambda i: (0, 0)),
            pl.BlockSpec((1, inner), lambda i: (0, 0)),
        ],
        out_specs=[
            pl.BlockSpec((pb, inner), lambda i: (i, 0)),
            pl.BlockSpec((pb, _K), lambda i: (i, 0)),
        ],
        out_shape=[
            jax.ShapeDtypeStruct((n, inner), jnp.float32),
            jax.ShapeDtypeStruct((n, _K), jnp.float32),
        ],
    )(h3, p2, a1, c1)


def _passc_body(h2_ref, wr_ref, ms_ref, z_ref, acc_ref, *, inner):
    i = pl.program_id(0)
    hw = h2_ref[...] * wr_ref[...]
    z = jnp.dot(ms_ref[...], hw, preferred_element_type=jnp.float32)
    z_ref[...] = z
    contrib = jnp.concatenate(
        [jnp.sum(z, axis=0, keepdims=True),
         jnp.sum(z * z, axis=0, keepdims=True),
         jnp.zeros((6, inner), jnp.float32)], axis=0)

    @pl.when(i == 0)
    def _():
        acc_ref[...] = jnp.zeros_like(acc_ref)

    acc_ref[...] += contrib


def _passc(h2, wr, msum, n, inner):
    pb = 128 if n % 128 == 0 else n
    return pl.pallas_call(
        functools.partial(_passc_body, inner=inner),
        grid=(n // pb,),
        in_specs=[
            pl.BlockSpec((pb * _K, inner), lambda i: (i, 0)),
            pl.BlockSpec((pb * _K, 1), lambda i: (i, 0)),
            pl.BlockSpec((pb, pb * _K), lambda i: (0, 0)),
        ],
        out_specs=[
            pl.BlockSpec((pb, inner), lambda i: (i, 0)),
            pl.BlockSpec((8, inner), lambda i: (0, 0)),
        ],
        out_shape=[
            jax.ShapeDtypeStruct((n, inner), jnp.float32),
            jax.ShapeDtypeStruct((8, inner), jnp.float32),
        ],
    )(h2, wr, msum)


def _passd1_body(z_ref, f3_ref, a2_ref, c2_ref, l1aT_ref, b1_ref,
                 t_ref, acc_ref, *, cin):
    i = pl.program_id(0)
    r = jnp.maximum(a2_ref[...] * z_ref[...] + c2_ref[...], 0.0)
    t = (jnp.dot(r, l1aT_ref[...], preferred_element_type=jnp.float32)
         + f3_ref[...] + b1_ref[...])
    t_ref[...] = t
    contrib = jnp.concatenate(
        [jnp.sum(t, axis=0, keepdims=True),
         jnp.sum(t * t, axis=0, keepdims=True),
         jnp.zeros((6, cin), jnp.float32)], axis=0)

    @pl.when(i == 0)
    def _():
        acc_ref[...] = jnp.zeros_like(acc_ref)

    acc_ref[...] += contrib


def _passd1(z, f3, a2, c2, l1aT, b1, n, cin, inner):
    rbg = 512 if n % 512 == 0 else n
    return pl.pallas_call(
        functools.partial(_passd1_body, cin=cin),
        grid=(n // rbg,),
        in_specs=[
            pl.BlockSpec((rbg, inner), lambda i: (i, 0)),
            pl.BlockSpec((rbg, cin), lambda i: (i, 0)),
            pl.BlockSpec((1, inner), lambda i: (0, 0)),
            pl.BlockSpec((1, inner), lambda i: (0, 0)),
            pl.BlockSpec((inner, cin), lambda i: (0, 0)),
            pl.BlockSpec((1, cin), lambda i: (0, 0)),
        ],
        out_specs=[
            pl.BlockSpec((rbg, cin), lambda i: (i, 0)),
            pl.BlockSpec((8, cin), lambda i: (0, 0)),
        ],
        out_shape=[
            jax.ShapeDtypeStruct((n, cin), jnp.float32),
            jax.ShapeDtypeStruct((8, cin), jnp.float32),
        ],
    )(z, f3, a2, c2, l1aT, b1)


def _passd2_body(t_ref, a3_ref, c3_ref, l2T_ref, b2_ref, o_ref):
    r = jnp.maximum(a3_ref[...] * t_ref[...] + c3_ref[...], 0.0)
    o_ref[...] = (jnp.dot(r, l2T_ref[...], preferred_element_type=jnp.float32)
                  + b2_ref[...])


def _passd2(t, a3, c3, l2T, b2, n, cin):
    rbg = 512 if n % 512 == 0 else n
    return pl.pallas_call(
        _passd2_body,
        grid=(n // rbg,),
        in_specs=[
            pl.BlockSpec((rbg, cin), lambda i: (i, 0)),
            pl.BlockSpec((1, cin), lambda i: (0, 0)),
            pl.BlockSpec((1, cin), lambda i: (0, 0)),
            pl.BlockSpec((cin, cin), lambda i: (0, 0)),
            pl.BlockSpec((1, cin), lambda i: (0, 0)),
        ],
        out_specs=pl.BlockSpec((rbg, cin), lambda i: (i, 0)),
        out_shape=jax.ShapeDtypeStruct((n, cin), jnp.float32),
    )(t, a3, c3, l2T, b2)


def kernel(coords, points, feature, conv_w, bn1_g, bn1_b, bn2_g, bn2_b,
           lin1_w, lin1_b, bn3_g, bn3_b, lin2_w, lin2_b):
    n, cin = feature.shape
    inner = conv_w.shape[0]
    f32 = jnp.float32
    b = coords[:, 3].astype(jnp.int32)

    ptsT = jnp.concatenate([points.T, jnp.zeros((5, n), f32)], axis=0)
    rb = 256 if n % 256 == 0 else n
    ct = 2048 if n % 2048 == 0 else n
    tr = _tile_ranges(b, n, rb, ct)
    nh = n // 2
    b2d = b[None, :]
    bcol = b[:, None]
    idx_lo = _knn(ptsT, b2d, bcol, tr, n, 0, nh)
    idx_hi = _knn(ptsT, b2d, bcol, tr, n, nh // rb, nh)

    x_cat = jnp.concatenate([feature, points], axis=1)
    pts_pad = jnp.concatenate([points, jnp.zeros((n, 5), f32)], axis=1)
    w3T = jnp.concatenate(
        [conv_w[:, cin:].T, jnp.zeros((5, inner), f32)], axis=0)
    g, p2, f3 = _gmat(x_cat, pts_pad, feature, conv_w.T, w3T,
                      lin1_w[:, inner:].T, n, cin, inner)

    pb = 128 if n % 128 == 0 else n
    msum = (jnp.arange(pb, dtype=jnp.int32)[:, None]
            == (jnp.arange(pb * _K, dtype=jnp.int32)[None, :] // _K)
            ).astype(f32)
    h_lo = _gather_rows(g, idx_lo.reshape(-1))
    h_hi = _gather_rows(g, idx_hi.reshape(-1))
    p2_lo, p2_hi = p2[:nh], p2[nh:]
    s = (_stats1(h_lo, p2_lo, msum, nh, inner)
         + _stats1(h_hi, p2_hi, msum, nh, inner))
    nk = jnp.float32(n * _K)
    mean1 = (s[0] - _K * s[3]) / nk
    ex2 = (s[1] - 2.0 * s[2] + _K * s[4]) / nk
    var1 = ex2 - mean1 * mean1
    a1 = bn1_g / jnp.sqrt(var1 + _EPS)
    c1 = bn1_b - a1 * mean1

    y_lo, w_lo = _passb(h_lo.reshape(nh, _K, inner), p2_lo,
                        a1[None], c1[None], nh, inner)
    y_hi, w_hi = _passb(h_hi.reshape(nh, _K, inner), p2_hi,
                        a1[None], c1[None], nh, inner)
    y = jnp.concatenate([y_lo, y_hi], axis=0)

    h2_lo = _gather_rows(y, idx_lo.reshape(-1))
    h2_hi = _gather_rows(y, idx_hi.reshape(-1))
    z_lo, acc2_lo = _passc(h2_lo, w_lo.reshape(nh * _K, 1), msum, nh, inner)
    z_hi, acc2_hi = _passc(h2_hi, w_hi.reshape(nh * _K, 1), msum, nh, inner)
    z = jnp.concatenate([z_lo, z_hi], axis=0)
    acc2 = acc2_lo + acc2_hi
    mean2 = acc2[0] / n
    var2 = acc2[1] / n - mean2 * mean2
    a2 = bn2_g / jnp.sqrt(var2 + _EPS)
    c2 = bn2_b - a2 * mean2

    t, acc3 = _passd1(z, f3, a2[None], c2[None], lin1_w[:, :inner].T,
                      lin1_b[None], n, cin, inner)
    mean3 = acc3[0] / n
    var3 = acc3[1] / n - mean3 * mean3
    a3 = bn3_g / jnp.sqrt(var3 + _EPS)
    c3 = bn3_b - a3 * mean3

    return _passd2(t, a3[None], c3[None], lin2_w.T, lin2_b[None], n, cin)


# speedup vs baseline: 1.3559x; 1.0038x over previous
import functools

import jax
import jax.numpy as jnp
from jax import lax
from jax.experimental import pallas as pl
from jax.experimental.pallas import tpu as pltpu
from jax.experimental.pallas import tpu_sc as plsc

_EPS = 1e-5
_K = 16
_BIG = 2**30


def _knn_body(tr_ref, ptsT_ref, b2d_ref, bcol_ref, idx_ref, d2_ref,
              *, rb, ct, n, i0):
    i = pl.program_id(0) + i0
    tlo = tr_ref[2 * i]
    thi = tr_ref[2 * i + 1]
    prow = ptsT_ref[:, pl.ds(i * rb, rb)]
    sqrow = lax.dot_general(
        prow * prow, jnp.ones((8, 1), jnp.float32),
        (((0,), (0,)), ((), ())), preferred_element_type=jnp.float32)
    bcol = bcol_ref[...]
    m0 = jnp.full((rb, 1), jnp.inf, jnp.float32)
    a0 = jnp.full((rb, 1), _BIG, jnp.int32)
    iota_c = lax.broadcasted_iota(jnp.int32, (rb, ct), 1)
    lane_k = lax.broadcasted_iota(jnp.int32, (rb, _K), 1)

    def lexlt(v1, i1, v2, i2):
        return (v1 < v2) | ((v1 == v2) & (i1 < i2))

    def lexmin(v1, i1, v2, i2):
        p = lexlt(v1, i1, v2, i2)
        return jnp.where(p, v1, v2), jnp.where(p, i1, i2)

    def merge2(c, mt1, cc1, mt2, cc2):
        M1, A1, M2, A2 = c
        p = lexlt(M1, A1, mt1, cc1)
        f_v = jnp.where(p, M1, mt1)
        f_i = jnp.where(p, A1, cc1)
        sa_v, sa_i = lexmin(M2, A2, mt1, cc1)
        sb_v, sb_i = lexmin(M1, A1, mt2, cc2)
        s_v = jnp.where(p, sa_v, sb_v)
        s_i = jnp.where(p, sa_i, sb_i)
        return f_v, f_i, s_v, s_i

    def top2_of_tile(c, tile, it):
        mt1 = jnp.min(tile, axis=1, keepdims=True)
        cc1 = jnp.min(jnp.where(tile == mt1, it, _BIG), axis=1, keepdims=True)
        tile2 = jnp.where(it == cc1, jnp.inf, tile)
        mt2 = jnp.min(tile2, axis=1, keepdims=True)
        cc2 = jnp.min(jnp.where(tile2 == mt2, it, _BIG), axis=1, keepdims=True)
        return merge2(c, mt1, cc1, mt2, cc2)

    def init_body(t, carry):
        off = pl.multiple_of(t * ct, ct)
        ptile = ptsT_ref[:, pl.ds(off, ct)]
        pp = lax.dot_general(prow, ptile, (((0,), (0,)), ((), ())),
                             preferred_element_type=jnp.float32)
        sqcol = jnp.sum(ptile * ptile, axis=0, keepdims=True)
        d2 = sqrow + sqcol - 2.0 * pp
        same = bcol == b2d_ref[:, pl.ds(off, ct)]
        tile = jnp.where(same, d2, jnp.inf)
        d2_ref[:, pl.ds(off, ct)] = tile
        return top2_of_tile(carry, tile, iota_c + t * ct)

    c0 = (m0, a0, m0, a0)
    _, a1g, _, a2g = lax.fori_loop(tlo, thi, init_body, c0)
    acc = jnp.where(lane_k == 0, a1g, 0)
    acc = jnp.where(lane_k == 1, a2g, acc)

    def round_body(r, carry):
        acc, p1, p2 = carry

        def sbody(t, c):
            off = pl.multiple_of(t * ct, ct)
            tile = d2_ref[:, pl.ds(off, ct)]
            it = iota_c + t * ct
            tile = jnp.where((it == p1) | (it == p2), jnp.inf, tile)
            d2_ref[:, pl.ds(off, ct)] = tile
            return top2_of_tile(c, tile, it)

        _, b1, _, b2 = lax.fori_loop(tlo, thi, sbody, c0)
        acc = jnp.where(lane_k == 2 * r, b1, acc)
        acc = jnp.where(lane_k == 2 * r + 1, b2, acc)
        return acc, b1, b2

    acc, _, _ = lax.fori_loop(1, _K // 2, round_body, (acc, a1g, a2g))
    idx_ref[...] = acc


def _knn(ptsT, b2d, bcol, tr, n, i0, nout):
    rb = 256 if n % 256 == 0 else n
    ct = 2048 if n % 2048 == 0 else n
    grid_spec = pltpu.PrefetchScalarGridSpec(
        num_scalar_prefetch=1,
        grid=(nout // rb,),
        in_specs=[
            pl.BlockSpec((8, n), lambda i, *_: (0, 0)),
            pl.BlockSpec((1, n), lambda i, *_: (0, 0)),
            pl.BlockSpec((rb, 1), lambda i, *_: (i + i0, 0)),
        ],
        out_specs=pl.BlockSpec((rb, _K), lambda i, *_: (i, 0)),
        scratch_shapes=[pltpu.VMEM((rb, n), jnp.float32)],
    )
    return pl.pallas_call(
        functools.partial(_knn_body, rb=rb, ct=ct, n=n, i0=i0),
        grid_spec=grid_spec,
        out_shape=jax.ShapeDtypeStruct((nout, _K), jnp.int32),
    )(tr, ptsT, b2d, bcol)


def _tile_ranges(b, n, rb, ct):
    nblk = n // rb
    ntiles = n // ct
    bb = b.reshape(nblk, rb)
    bcast = b[None, :]
    lo_col = jnp.sum((bcast < bb[:, 0][:, None]).astype(jnp.int32), axis=1)
    hi_col = jnp.sum((bcast <= bb[:, -1][:, None]).astype(jnp.int32), axis=1)
    vals = jnp.arange(8, dtype=jnp.int32)
    hist = jnp.sum((bcast == vals[:, None]).astype(jnp.int32), axis=1)
    minsz = jnp.min(jnp.where(hist > 0, hist, n))
    tlo = jnp.where(minsz < _K, 0, lo_col // ct)
    thi = jnp.where(minsz < _K, ntiles, (hi_col + ct - 1) // ct)
    return jnp.stack([tlo, thi], axis=1).reshape(-1).astype(jnp.int32)


def _gmat_body(x_ref, pts_ref, feat_ref, cwT_ref, w3T_ref, l1bT_ref,
               g_ref, p2_ref, f3_ref):
    g_ref[...] = jnp.dot(x_ref[...], cwT_ref[...],
                         preferred_element_type=jnp.float32)
    p2_ref[...] = jnp.dot(pts_ref[...], w3T_ref[...],
                          preferred_element_type=jnp.float32)
    f3_ref[...] = jnp.dot(feat_ref[...], l1bT_ref[...],
                          preferred_element_type=jnp.float32)


def _gmat(x_cat, pts_pad, feature, cwT, w3T, l1bT, n, cin, inner):
    rbg = 512 if n % 512 == 0 else n
    c3 = cin + 3
    return pl.pallas_call(
        _gmat_body,
        grid=(n // rbg,),
        in_specs=[
            pl.BlockSpec((rbg, c3), lambda i: (i, 0)),
            pl.BlockSpec((rbg, 8), lambda i: (i, 0)),
            pl.BlockSpec((rbg, cin), lambda i: (i, 0)),
            pl.BlockSpec((c3, inner), lambda i: (0, 0)),
            pl.BlockSpec((8, inner), lambda i: (0, 0)),
            pl.BlockSpec((cin, inner), lambda i: (0, 0)),
        ],
        out_specs=[
            pl.BlockSpec((rbg, inner), lambda i: (i, 0)),
            pl.BlockSpec((rbg, inner), lambda i: (i, 0)),
            pl.BlockSpec((rbg, inner), lambda i: (i, 0)),
        ],
        out_shape=[
            jax.ShapeDtypeStruct((n, inner), jnp.float32),
            jax.ShapeDtypeStruct((n, inner), jnp.float32),
            jax.ShapeDtypeStruct((n, inner), jnp.float32),
        ],
    )(x_cat, pts_pad, feature, cwT, w3T, l1bT)


def _gather_rows(table, idx_flat):
    nrows = idx_flat.shape[0]
    d = table.shape[1]
    nw = 32
    per_w = nrows // nw
    ch = 128
    nch = per_w // ch
    mesh = plsc.VectorSubcoreMesh(core_axis_name="c", subcore_axis_name="s")

    @functools.partial(
        pl.kernel, mesh=mesh,
        out_type=jax.ShapeDtypeStruct((nrows, d), jnp.float32),
        scratch_types=[
            pltpu.VMEM((per_w,), jnp.int32),
            pltpu.VMEM((ch, d), jnp.float32),
            pltpu.VMEM((ch, d), jnp.float32),
            pltpu.SemaphoreType.DMA,
            pltpu.SemaphoreType.DMA,
            pltpu.SemaphoreType.DMA,
            pltpu.SemaphoreType.DMA,
        ],
    )
    def gk(table_hbm, idx_hbm, out_hbm, idx_all, rows0, rows1,
           sem0, sem1, osem0, osem1):
        wid = lax.axis_index("s") * 2 + lax.axis_index("c")
        base = wid * per_w
        rows_v = [rows0, rows1]
        sems = [sem0, sem1]
        osems = [osem0, osem1]
        pltpu.sync_copy(idx_hbm.at[pl.ds(base, per_w)], idx_all)

        def start(c, slot):
            pltpu.async_copy(
                table_hbm.at[idx_all.at[pl.ds(c * ch, ch)]],
                rows_v[slot], sems[slot])

        def handle(c, slot):
            pltpu.make_async_copy(
                table_hbm.at[idx_all.at[pl.ds(c * ch, ch)]],
                rows_v[slot], sems[slot]).wait()
            pltpu.async_copy(rows_v[slot],
                             out_hbm.at[pl.ds(base + c * ch, ch)],
                             osems[slot])

        def owait(c, slot):
            pltpu.make_async_copy(
                rows_v[slot], out_hbm.at[pl.ds(base + c * ch, ch)],
                osems[slot]).wait()

        start(0, 0)

        def body(c, carry):
            slot = lax.rem(c, 2)

            @pl.when(c >= 1)
            def _():
                jax.lax.switch(1 - slot, [lambda: owait(c - 1, 0),
                                          lambda: owait(c - 1, 1)])

            @pl.when(c + 1 < nch)
            def _():
                jax.lax.switch(1 - slot, [lambda: start(c + 1, 0),
                                          lambda: start(c + 1, 1)])

            jax.lax.switch(slot, [lambda: handle(c, 0), lambda: handle(c, 1)])
            return carry

        lax.fori_loop(0, nch, body, 0)
        owait(nch - 1, (nch - 1) % 2)

    return gk(table, idx_flat)


def _stats1_body(h_ref, p2_ref, ms_ref, out_ref, *, inner):
    i = pl.program_id(0)
    h = h_ref[...]
    p2 = p2_ref[...]
    hs = jnp.dot(ms_ref[...], h, preferred_element_type=jnp.float32)
    s1 = jnp.sum(h, axis=0, keepdims=True)
    s2 = jnp.sum(h * h, axis=0, keepdims=True)
    s3 = jnp.sum(p2 * hs, axis=0, keepdims=True)
    s4 = jnp.sum(p2, axis=0, keepdims=True)
    s5 = jnp.sum(p2 * p2, axis=0, keepdims=True)
    contrib = jnp.concatenate(
        [s1, s2, s3, s4, s5, jnp.zeros((3, inner), jnp.float32)], axis=0)

    @pl.when(i == 0)
    def _():
        out_ref[...] = jnp.zeros_like(out_ref)

    out_ref[...] += contrib


def _stats1(h, p2, msum, n, inner):
    pb = 128 if n % 128 == 0 else n
    return pl.pallas_call(
        functools.partial(_stats1_body, inner=inner),
        grid=(n // pb,),
        in_specs=[
            pl.BlockSpec((pb * _K, inner), lambda i: (i, 0)),
            pl.BlockSpec((pb, inner), lambda i: (i, 0)),
            pl.BlockSpec((pb, pb * _K), lambda i: (0, 0)),
        ],
        out_specs=pl.BlockSpec((8, inner), lambda i: (0, 0)),
        out_shape=jax.ShapeDtypeStruct((8, inner), jnp.float32),
    )(h, p2, msum)


def _passb_body(h_ref, p2_ref, a_ref, c_ref, y_ref, w_ref, *, pb, inner):
    a = a_ref[...]
    u = c_ref[...] - a * p2_ref[...]
    s = a * h_ref[:, 0, :] + u
    y = None
    wcols = []
    for k in range(_K):
        xk = a * h_ref[:, k, :] + u
        wk = jnp.sum(xk * s, axis=1, keepdims=True)
        wcols.append(wk)
        yk = xk * wk
        y = yk if y is None else y + yk
    w_ref[...] = jnp.concatenate(wcols, axis=1)
    y_ref[...] = y


def _passb(h3, p2, a1, c1, n, inner):
    pb = 128 if n % 128 == 0 else n
    return pl.pallas_call(
        functools.partial(_passb_body, pb=pb, inner=inner),
        grid=(n // pb,),
        in_specs=[
            pl.BlockSpec((pb, _K, inner), lambda i: (i, 0, 0)),
            pl.BlockSpec((pb, inner), lambda i: (i, 0)),
            pl.BlockSpec((1, inner), lambda i: (0, 0)),
            pl.BlockSpec((1, inner), lambda i: (0, 0)),
        ],
        out_specs=[
            pl.BlockSpec((pb, inner), lambda i: (i, 0)),
            pl.BlockSpec((pb, _K), lambda i: (i, 0)),
        ],
        out_shape=[
            jax.ShapeDtypeStruct((n, inner), jnp.float32),
            jax.ShapeDtypeStruct((n, _K), jnp.float32),
        ],
    )(h3, p2, a1, c1)


def _passc_body(h2_ref, wr_ref, ms_ref, z_ref, acc_ref, *, inner):
    i = pl.program_id(0)
    hw = h2_ref[...] * wr_ref[...]
    z = jnp.dot(ms_ref[...], hw, preferred_element_type=jnp.float32)
    z_ref[...] = z
    contrib = jnp.concatenate(
        [jnp.sum(z, axis=0, keepdims=True),
         jnp.sum(z * z, axis=0, keepdims=True),
         jnp.zeros((6, inner), jnp.float32)], axis=0)

    @pl.when(i == 0)
    def _():
        acc_ref[...] = jnp.zeros_like(acc_ref)

    acc_ref[...] += contrib


def _passc(h2, wr, msum, n, inner):
    pb = 128 if n % 128 == 0 else n
    return pl.pallas_call(
        functools.partial(_passc_body, inner=inner),
        grid=(n // pb,),
        in_specs=[
            pl.BlockSpec((pb * _K, inner), lambda i: (i, 0)),
            pl.BlockSpec((pb * _K, 1), lambda i: (i, 0)),
            pl.BlockSpec((pb, pb * _K), lambda i: (0, 0)),
        ],
        out_specs=[
            pl.BlockSpec((pb, inner), lambda i: (i, 0)),
            pl.BlockSpec((8, inner), lambda i: (0, 0)),
        ],
        out_shape=[
            jax.ShapeDtypeStruct((n, inner), jnp.float32),
            jax.ShapeDtypeStruct((8, inner), jnp.float32),
        ],
    )(h2, wr, msum)


def _passd1_body(zlo_ref, zhi_ref, f3_ref, a2_ref, c2_ref, l1aT_ref, b1_ref,
                 t_ref, acc_ref, *, cin, nbh):
    i = pl.program_id(0)
    z = jnp.where(i < nbh, zlo_ref[...], zhi_ref[...])
    r = jnp.maximum(a2_ref[...] * z + c2_ref[...], 0.0)
    t = (jnp.dot(r, l1aT_ref[...], preferred_element_type=jnp.float32)
         + f3_ref[...] + b1_ref[...])
    t_ref[...] = t
    contrib = jnp.concatenate(
        [jnp.sum(t, axis=0, keepdims=True),
         jnp.sum(t * t, axis=0, keepdims=True),
         jnp.zeros((6, cin), jnp.float32)], axis=0)

    @pl.when(i == 0)
    def _():
        acc_ref[...] = jnp.zeros_like(acc_ref)

    acc_ref[...] += contrib


def _passd1(z_lo, z_hi, f3, a2, c2, l1aT, b1, n, cin, inner):
    rbg = 512 if n % 512 == 0 else n
    nbh = (n // 2) // rbg
    return pl.pallas_call(
        functools.partial(_passd1_body, cin=cin, nbh=nbh),
        grid=(n // rbg,),
        in_specs=[
            pl.BlockSpec((rbg, inner), lambda i: (jnp.minimum(i, nbh - 1), 0)),
            pl.BlockSpec((rbg, inner),
                         lambda i: (jnp.maximum(i - nbh, 0), 0)),
            pl.BlockSpec((rbg, cin), lambda i: (i, 0)),
            pl.BlockSpec((1, inner), lambda i: (0, 0)),
            pl.BlockSpec((1, inner), lambda i: (0, 0)),
            pl.BlockSpec((inner, cin), lambda i: (0, 0)),
            pl.BlockSpec((1, cin), lambda i: (0, 0)),
        ],
        out_specs=[
            pl.BlockSpec((rbg, cin), lambda i: (i, 0)),
            pl.BlockSpec((8, cin), lambda i: (0, 0)),
        ],
        out_shape=[
            jax.ShapeDtypeStruct((n, cin), jnp.float32),
            jax.ShapeDtypeStruct((8, cin), jnp.float32),
        ],
    )(z_lo, z_hi, f3, a2, c2, l1aT, b1)


def _passd2_body(t_ref, a3_ref, c3_ref, l2T_ref, b2_ref, o_ref):
    r = jnp.maximum(a3_ref[...] * t_ref[...] + c3_ref[...], 0.0)
    o_ref[...] = (jnp.dot(r, l2T_ref[...], preferred_element_type=jnp.float32)
                  + b2_ref[...])


def _passd2(t, a3, c3, l2T, b2, n, cin):
    rbg = 512 if n % 512 == 0 else n
    return pl.pallas_call(
        _passd2_body,
        grid=(n // rbg,),
        in_specs=[
            pl.BlockSpec((rbg, cin), lambda i: (i, 0)),
            pl.BlockSpec((1, cin), lambda i: (0, 0)),
            pl.BlockSpec((1, cin), lambda i: (0, 0)),
            pl.BlockSpec((cin, cin), lambda i: (0, 0)),
            pl.BlockSpec((1, cin), lambda i: (0, 0)),
        ],
        out_specs=pl.BlockSpec((rbg, cin), lambda i: (i, 0)),
        out_shape=jax.ShapeDtypeStruct((n, cin), jnp.float32),
    )(t, a3, c3, l2T, b2)


def kernel(coords, points, feature, conv_w, bn1_g, bn1_b, bn2_g, bn2_b,
           lin1_w, lin1_b, bn3_g, bn3_b, lin2_w, lin2_b):
    n, cin = feature.shape
    inner = conv_w.shape[0]
    f32 = jnp.float32
    b = coords[:, 3].astype(jnp.int32)

    ptsT = jnp.concatenate([points.T, jnp.zeros((5, n), f32)], axis=0)
    rb = 256 if n % 256 == 0 else n
    ct = 2048 if n % 2048 == 0 else n
    tr = _tile_ranges(b, n, rb, ct)
    nh = n // 2
    b2d = b[None, :]
    bcol = b[:, None]
    idx_lo = _knn(ptsT, b2d, bcol, tr, n, 0, nh)
    idx_hi = _knn(ptsT, b2d, bcol, tr, n, nh // rb, nh)

    x_cat = jnp.concatenate([feature, points], axis=1)
    pts_pad = jnp.concatenate([points, jnp.zeros((n, 5), f32)], axis=1)
    w3T = jnp.concatenate(
        [conv_w[:, cin:].T, jnp.zeros((5, inner), f32)], axis=0)
    g, p2, f3 = _gmat(x_cat, pts_pad, feature, conv_w.T, w3T,
                      lin1_w[:, inner:].T, n, cin, inner)

    pb = 128 if n % 128 == 0 else n
    msum = (jnp.arange(pb, dtype=jnp.int32)[:, None]
            == (jnp.arange(pb * _K, dtype=jnp.int32)[None, :] // _K)
            ).astype(f32)
    h_lo = _gather_rows(g, idx_lo.reshape(-1))
    h_hi = _gather_rows(g, idx_hi.reshape(-1))
    p2_lo, p2_hi = p2[:nh], p2[nh:]
    s = (_stats1(h_lo, p2_lo, msum, nh, inner)
         + _stats1(h_hi, p2_hi, msum, nh, inner))
    nk = jnp.float32(n * _K)
    mean1 = (s[0] - _K * s[3]) / nk
    ex2 = (s[1] - 2.0 * s[2] + _K * s[4]) / nk
    var1 = ex2 - mean1 * mean1
    a1 = bn1_g / jnp.sqrt(var1 + _EPS)
    c1 = bn1_b - a1 * mean1

    y_lo, w_lo = _passb(h_lo.reshape(nh, _K, inner), p2_lo,
                        a1[None], c1[None], nh, inner)
    y_hi, w_hi = _passb(h_hi.reshape(nh, _K, inner), p2_hi,
                        a1[None], c1[None], nh, inner)
    y = jnp.concatenate([y_lo, y_hi], axis=0)

    h2_lo = _gather_rows(y, idx_lo.reshape(-1))
    h2_hi = _gather_rows(y, idx_hi.reshape(-1))
    z_lo, acc2_lo = _passc(h2_lo, w_lo.reshape(nh * _K, 1), msum, nh, inner)
    z_hi, acc2_hi = _passc(h2_hi, w_hi.reshape(nh * _K, 1), msum, nh, inner)
    acc2 = acc2_lo + acc2_hi
    mean2 = acc2[0] / n
    var2 = acc2[1] / n - mean2 * mean2
    a2 = bn2_g / jnp.sqrt(var2 + _EPS)
    c2 = bn2_b - a2 * mean2

    t, acc3 = _passd1(z_lo, z_hi, f3, a2[None], c2[None], lin1_w[:, :inner].T,
                      lin1_b[None], n, cin, inner)
    mean3 = acc3[0] / n
    var3 = acc3[1] / n - mean3 * mean3
    a3 = bn3_g / jnp.sqrt(var3 + _EPS)
    c3 = bn3_b - a3 * mean3

    return _passd2(t, a3[None], c3[None], lin2_w.T, lin2_b[None], n, cin)


# speedup vs baseline: 1.3921x; 1.0267x over previous
import functools

import jax
import jax.numpy as jnp
from jax import lax
from jax.experimental import pallas as pl
from jax.experimental.pallas import tpu as pltpu
from jax.experimental.pallas import tpu_sc as plsc

_EPS = 1e-5
_K = 16
_BIG = 2**30


def _knn_body(tr_ref, ptsT_ref, b2d_ref, bcol_ref, idx_ref, d2_ref,
              *, rb, ct, n, i0):
    i = pl.program_id(0) + i0
    tlo = tr_ref[2 * i]
    thi = tr_ref[2 * i + 1]
    prow = ptsT_ref[:, pl.ds(i * rb, rb)]
    sqrow = lax.dot_general(
        prow * prow, jnp.ones((8, 1), jnp.float32),
        (((0,), (0,)), ((), ())), preferred_element_type=jnp.float32)
    bcol = bcol_ref[...]
    m0 = jnp.full((rb, 1), jnp.inf, jnp.float32)
    a0 = jnp.full((rb, 1), _BIG, jnp.int32)
    iota_c = lax.broadcasted_iota(jnp.int32, (rb, ct), 1)
    lane_k = lax.broadcasted_iota(jnp.int32, (rb, _K), 1)

    def lexlt(v1, i1, v2, i2):
        return (v1 < v2) | ((v1 == v2) & (i1 < i2))

    def lexmin(v1, i1, v2, i2):
        p = lexlt(v1, i1, v2, i2)
        return jnp.where(p, v1, v2), jnp.where(p, i1, i2)

    def merge2(c, mt1, cc1, mt2, cc2):
        M1, A1, M2, A2 = c
        p = lexlt(M1, A1, mt1, cc1)
        f_v = jnp.where(p, M1, mt1)
        f_i = jnp.where(p, A1, cc1)
        sa_v, sa_i = lexmin(M2, A2, mt1, cc1)
        sb_v, sb_i = lexmin(M1, A1, mt2, cc2)
        s_v = jnp.where(p, sa_v, sb_v)
        s_i = jnp.where(p, sa_i, sb_i)
        return f_v, f_i, s_v, s_i

    def top2_of_tile(c, tile, it):
        mt1 = jnp.min(tile, axis=1, keepdims=True)
        cc1 = jnp.min(jnp.where(tile == mt1, it, _BIG), axis=1, keepdims=True)
        tile2 = jnp.where(it == cc1, jnp.inf, tile)
        mt2 = jnp.min(tile2, axis=1, keepdims=True)
        cc2 = jnp.min(jnp.where(tile2 == mt2, it, _BIG), axis=1, keepdims=True)
        return merge2(c, mt1, cc1, mt2, cc2)

    def init_body(t, carry):
        off = pl.multiple_of(t * ct, ct)
        ptile = ptsT_ref[:, pl.ds(off, ct)]
        pp = lax.dot_general(prow, ptile, (((0,), (0,)), ((), ())),
                             preferred_element_type=jnp.float32)
        sqcol = jnp.sum(ptile * ptile, axis=0, keepdims=True)
        d2 = sqrow + sqcol - 2.0 * pp
        same = bcol == b2d_ref[:, pl.ds(off, ct)]
        tile = jnp.where(same, d2, jnp.inf)
        d2_ref[:, pl.ds(off, ct)] = tile
        return top2_of_tile(carry, tile, iota_c + t * ct)

    c0 = (m0, a0, m0, a0)
    _, a1g, _, a2g = lax.fori_loop(tlo, thi, init_body, c0)
    acc = jnp.where(lane_k == 0, a1g, 0)
    acc = jnp.where(lane_k == 1, a2g, acc)

    def round_body(r, carry):
        acc, p1, p2 = carry

        def sbody(t, c):
            off = pl.multiple_of(t * ct, ct)
            tile = d2_ref[:, pl.ds(off, ct)]
            it = iota_c + t * ct
            tile = jnp.where((it == p1) | (it == p2), jnp.inf, tile)
            d2_ref[:, pl.ds(off, ct)] = tile
            return top2_of_tile(c, tile, it)

        _, b1, _, b2 = lax.fori_loop(tlo, thi, sbody, c0)
        acc = jnp.where(lane_k == 2 * r, b1, acc)
        acc = jnp.where(lane_k == 2 * r + 1, b2, acc)
        return acc, b1, b2

    acc, _, _ = lax.fori_loop(1, _K // 2, round_body, (acc, a1g, a2g))
    idx_ref[...] = acc


def _knn(ptsT, b2d, bcol, tr, n, i0, nout):
    rb = 256 if n % 256 == 0 else n
    ct = 2048 if n % 2048 == 0 else n
    grid_spec = pltpu.PrefetchScalarGridSpec(
        num_scalar_prefetch=1,
        grid=(nout // rb,),
        in_specs=[
            pl.BlockSpec((8, n), lambda i, *_: (0, 0)),
            pl.BlockSpec((1, n), lambda i, *_: (0, 0)),
            pl.BlockSpec((rb, 1), lambda i, *_: (i + i0, 0)),
        ],
        out_specs=pl.BlockSpec((rb, _K), lambda i, *_: (i, 0)),
        scratch_shapes=[pltpu.VMEM((rb, n), jnp.float32)],
    )
    return pl.pallas_call(
        functools.partial(_knn_body, rb=rb, ct=ct, n=n, i0=i0),
        grid_spec=grid_spec,
        out_shape=jax.ShapeDtypeStruct((nout, _K), jnp.int32),
    )(tr, ptsT, b2d, bcol)


def _tile_ranges(b, n, rb, ct):
    nblk = n // rb
    ntiles = n // ct
    bb = b.reshape(nblk, rb)
    bcast = b[None, :]
    lo_col = jnp.sum((bcast < bb[:, 0][:, None]).astype(jnp.int32), axis=1)
    hi_col = jnp.sum((bcast <= bb[:, -1][:, None]).astype(jnp.int32), axis=1)
    vals = jnp.arange(8, dtype=jnp.int32)
    hist = jnp.sum((bcast == vals[:, None]).astype(jnp.int32), axis=1)
    minsz = jnp.min(jnp.where(hist > 0, hist, n))
    tlo = jnp.where(minsz < _K, 0, lo_col // ct)
    thi = jnp.where(minsz < _K, ntiles, (hi_col + ct - 1) // ct)
    return jnp.stack([tlo, thi], axis=1).reshape(-1).astype(jnp.int32)


def _gmat_body(x_ref, pts_ref, feat_ref, cwT_ref, w3T_ref, l1bT_ref,
               g_ref, p2_ref, f3_ref):
    g_ref[...] = jnp.dot(x_ref[...], cwT_ref[...],
                         preferred_element_type=jnp.float32)
    p2_ref[...] = jnp.dot(pts_ref[...], w3T_ref[...],
                          preferred_element_type=jnp.float32)
    f3_ref[...] = jnp.dot(feat_ref[...], l1bT_ref[...],
                          preferred_element_type=jnp.float32)


def _gmat(x_cat, pts_pad, feature, cwT, w3T, l1bT, n, cin, inner):
    rbg = 512 if n % 512 == 0 else n
    c3 = cin + 3
    return pl.pallas_call(
        _gmat_body,
        grid=(n // rbg,),
        in_specs=[
            pl.BlockSpec((rbg, c3), lambda i: (i, 0)),
            pl.BlockSpec((rbg, 8), lambda i: (i, 0)),
            pl.BlockSpec((rbg, cin), lambda i: (i, 0)),
            pl.BlockSpec((c3, inner), lambda i: (0, 0)),
            pl.BlockSpec((8, inner), lambda i: (0, 0)),
            pl.BlockSpec((cin, inner), lambda i: (0, 0)),
        ],
        out_specs=[
            pl.BlockSpec((rbg, inner), lambda i: (i, 0)),
            pl.BlockSpec((rbg, inner), lambda i: (i, 0)),
            pl.BlockSpec((rbg, inner), lambda i: (i, 0)),
        ],
        out_shape=[
            jax.ShapeDtypeStruct((n, inner), jnp.float32),
            jax.ShapeDtypeStruct((n, inner), jnp.float32),
            jax.ShapeDtypeStruct((n, inner), jnp.float32),
        ],
    )(x_cat, pts_pad, feature, cwT, w3T, l1bT)


def _gather_rows(table, idx_flat):
    nrows = idx_flat.shape[0]
    d = table.shape[1]
    nw = 32
    per_w = nrows // nw
    ch = 128
    nch = per_w // ch
    mesh = plsc.VectorSubcoreMesh(core_axis_name="c", subcore_axis_name="s")

    @functools.partial(
        pl.kernel, mesh=mesh,
        out_type=jax.ShapeDtypeStruct((nrows, d), jnp.float32),
        scratch_types=[
            pltpu.VMEM((per_w,), jnp.int32),
            pltpu.VMEM((ch, d), jnp.float32),
            pltpu.VMEM((ch, d), jnp.float32),
            pltpu.SemaphoreType.DMA,
            pltpu.SemaphoreType.DMA,
            pltpu.SemaphoreType.DMA,
            pltpu.SemaphoreType.DMA,
        ],
    )
    def gk(table_hbm, idx_hbm, out_hbm, idx_all, rows0, rows1,
           sem0, sem1, osem0, osem1):
        wid = lax.axis_index("s") * 2 + lax.axis_index("c")
        base = wid * per_w
        rows_v = [rows0, rows1]
        sems = [sem0, sem1]
        osems = [osem0, osem1]
        pltpu.sync_copy(idx_hbm.at[pl.ds(base, per_w)], idx_all)

        def start(c, slot):
            pltpu.async_copy(
                table_hbm.at[idx_all.at[pl.ds(c * ch, ch)]],
                rows_v[slot], sems[slot])

        def handle(c, slot):
            pltpu.make_async_copy(
                table_hbm.at[idx_all.at[pl.ds(c * ch, ch)]],
                rows_v[slot], sems[slot]).wait()
            pltpu.async_copy(rows_v[slot],
                             out_hbm.at[pl.ds(base + c * ch, ch)],
                             osems[slot])

        def owait(c, slot):
            pltpu.make_async_copy(
                rows_v[slot], out_hbm.at[pl.ds(base + c * ch, ch)],
                osems[slot]).wait()

        start(0, 0)

        def body(c, carry):
            slot = lax.rem(c, 2)

            @pl.when(c >= 1)
            def _():
                jax.lax.switch(1 - slot, [lambda: owait(c - 1, 0),
                                          lambda: owait(c - 1, 1)])

            @pl.when(c + 1 < nch)
            def _():
                jax.lax.switch(1 - slot, [lambda: start(c + 1, 0),
                                          lambda: start(c + 1, 1)])

            jax.lax.switch(slot, [lambda: handle(c, 0), lambda: handle(c, 1)])
            return carry

        lax.fori_loop(0, nch, body, 0)
        owait(nch - 1, (nch - 1) % 2)

    return gk(table, idx_flat)


def _stats1_body(h_ref, p2_ref, out_ref, *, inner):
    i = pl.program_id(0)
    p2 = p2_ref[...]
    hs = None
    sq = None
    for k in range(_K):
        hk = h_ref[k, :, :]
        hs = hk if hs is None else hs + hk
        sq = hk * hk if sq is None else sq + hk * hk
    s1 = jnp.sum(hs, axis=0, keepdims=True)
    s2 = jnp.sum(sq, axis=0, keepdims=True)
    s3 = jnp.sum(p2 * hs, axis=0, keepdims=True)
    s4 = jnp.sum(p2, axis=0, keepdims=True)
    s5 = jnp.sum(p2 * p2, axis=0, keepdims=True)
    contrib = jnp.concatenate(
        [s1, s2, s3, s4, s5, jnp.zeros((3, inner), jnp.float32)], axis=0)

    @pl.when(i == 0)
    def _():
        out_ref[...] = jnp.zeros_like(out_ref)

    out_ref[...] += contrib


def _stats1(hkm, p2, n, inner):
    pb = 128 if n % 128 == 0 else n
    return pl.pallas_call(
        functools.partial(_stats1_body, inner=inner),
        grid=(n // pb,),
        in_specs=[
            pl.BlockSpec((_K, pb, inner), lambda i: (0, i, 0)),
            pl.BlockSpec((pb, inner), lambda i: (i, 0)),
        ],
        out_specs=pl.BlockSpec((8, inner), lambda i: (0, 0)),
        out_shape=jax.ShapeDtypeStruct((8, inner), jnp.float32),
    )(hkm, p2)


def _passb_body(h_ref, p2_ref, a_ref, c_ref, y_ref, w_ref, *, pb, inner):
    a = a_ref[...]
    u = c_ref[...] - a * p2_ref[...]
    s = a * h_ref[0, :, :] + u
    y = None
    wcols = []
    for k in range(_K):
        xk = a * h_ref[k, :, :] + u
        wk = jnp.sum(xk * s, axis=1, keepdims=True)
        wcols.append(wk)
        yk = xk * wk
        y = yk if y is None else y + yk
    w_ref[...] = jnp.concatenate(wcols, axis=1)
    y_ref[...] = y


def _passb(hkm, p2, a1, c1, n, inner):
    pb = 128 if n % 128 == 0 else n
    return pl.pallas_call(
        functools.partial(_passb_body, pb=pb, inner=inner),
        grid=(n // pb,),
        in_specs=[
            pl.BlockSpec((_K, pb, inner), lambda i: (0, i, 0)),
            pl.BlockSpec((pb, inner), lambda i: (i, 0)),
            pl.BlockSpec((1, inner), lambda i: (0, 0)),
            pl.BlockSpec((1, inner), lambda i: (0, 0)),
        ],
        out_specs=[
            pl.BlockSpec((pb, inner), lambda i: (i, 0)),
            pl.BlockSpec((pb, _K), lambda i: (i, 0)),
        ],
        out_shape=[
            jax.ShapeDtypeStruct((n, inner), jnp.float32),
            jax.ShapeDtypeStruct((n, _K), jnp.float32),
        ],
    )(hkm, p2, a1, c1)


def _passc_body(h2_ref, w_ref, z_ref, acc_ref, *, inner):
    i = pl.program_id(0)
    z = None
    for k in range(_K):
        zk = h2_ref[k, :, :] * w_ref[k, :, :]
        z = zk if z is None else z + zk
    z_ref[...] = z
    contrib = jnp.concatenate(
        [jnp.sum(z, axis=0, keepdims=True),
         jnp.sum(z * z, axis=0, keepdims=True),
         jnp.zeros((6, inner), jnp.float32)], axis=0)

    @pl.when(i == 0)
    def _():
        acc_ref[...] = jnp.zeros_like(acc_ref)

    acc_ref[...] += contrib


def _passc(h2km, wkm, n, inner):
    pb = 128 if n % 128 == 0 else n
    return pl.pallas_call(
        functools.partial(_passc_body, inner=inner),
        grid=(n // pb,),
        in_specs=[
            pl.BlockSpec((_K, pb, inner), lambda i: (0, i, 0)),
            pl.BlockSpec((_K, pb, 1), lambda i: (0, i, 0)),
        ],
        out_specs=[
            pl.BlockSpec((pb, inner), lambda i: (i, 0)),
            pl.BlockSpec((8, inner), lambda i: (0, 0)),
        ],
        out_shape=[
            jax.ShapeDtypeStruct((n, inner), jnp.float32),
            jax.ShapeDtypeStruct((8, inner), jnp.float32),
        ],
    )(h2km, wkm)


def _passd1_body(zlo_ref, zhi_ref, f3_ref, a2_ref, c2_ref, l1aT_ref, b1_ref,
                 t_ref, acc_ref, *, cin, nbh):
    i = pl.program_id(0)
    z = jnp.where(i < nbh, zlo_ref[...], zhi_ref[...])
    r = jnp.maximum(a2_ref[...] * z + c2_ref[...], 0.0)
    t = (jnp.dot(r, l1aT_ref[...], preferred_element_type=jnp.float32)
         + f3_ref[...] + b1_ref[...])
    t_ref[...] = t
    contrib = jnp.concatenate(
        [jnp.sum(t, axis=0, keepdims=True),
         jnp.sum(t * t, axis=0, keepdims=True),
         jnp.zeros((6, cin), jnp.float32)], axis=0)

    @pl.when(i == 0)
    def _():
        acc_ref[...] = jnp.zeros_like(acc_ref)

    acc_ref[...] += contrib


def _passd1(z_lo, z_hi, f3, a2, c2, l1aT, b1, n, cin, inner):
    rbg = 512 if n % 512 == 0 else n
    nbh = (n // 2) // rbg
    return pl.pallas_call(
        functools.partial(_passd1_body, cin=cin, nbh=nbh),
        grid=(n // rbg,),
        in_specs=[
            pl.BlockSpec((rbg, inner), lambda i: (jnp.minimum(i, nbh - 1), 0)),
            pl.BlockSpec((rbg, inner),
                         lambda i: (jnp.maximum(i - nbh, 0), 0)),
            pl.BlockSpec((rbg, cin), lambda i: (i, 0)),
            pl.BlockSpec((1, inner), lambda i: (0, 0)),
            pl.BlockSpec((1, inner), lambda i: (0, 0)),
            pl.BlockSpec((inner, cin), lambda i: (0, 0)),
            pl.BlockSpec((1, cin), lambda i: (0, 0)),
        ],
        out_specs=[
            pl.BlockSpec((rbg, cin), lambda i: (i, 0)),
            pl.BlockSpec((8, cin), lambda i: (0, 0)),
        ],
        out_shape=[
            jax.ShapeDtypeStruct((n, cin), jnp.float32),
            jax.ShapeDtypeStruct((8, cin), jnp.float32),
        ],
    )(z_lo, z_hi, f3, a2, c2, l1aT, b1)


def _passd2_body(t_ref, a3_ref, c3_ref, l2T_ref, b2_ref, o_ref):
    r = jnp.maximum(a3_ref[...] * t_ref[...] + c3_ref[...], 0.0)
    o_ref[...] = (jnp.dot(r, l2T_ref[...], preferred_element_type=jnp.float32)
                  + b2_ref[...])


def _passd2(t, a3, c3, l2T, b2, n, cin):
    rbg = 512 if n % 512 == 0 else n
    return pl.pallas_call(
        _passd2_body,
        grid=(n // rbg,),
        in_specs=[
            pl.BlockSpec((rbg, cin), lambda i: (i, 0)),
            pl.BlockSpec((1, cin), lambda i: (0, 0)),
            pl.BlockSpec((1, cin), lambda i: (0, 0)),
            pl.BlockSpec((cin, cin), lambda i: (0, 0)),
            pl.BlockSpec((1, cin), lambda i: (0, 0)),
        ],
        out_specs=pl.BlockSpec((rbg, cin), lambda i: (i, 0)),
        out_shape=jax.ShapeDtypeStruct((n, cin), jnp.float32),
    )(t, a3, c3, l2T, b2)


def kernel(coords, points, feature, conv_w, bn1_g, bn1_b, bn2_g, bn2_b,
           lin1_w, lin1_b, bn3_g, bn3_b, lin2_w, lin2_b):
    n, cin = feature.shape
    inner = conv_w.shape[0]
    f32 = jnp.float32
    b = coords[:, 3].astype(jnp.int32)

    ptsT = jnp.concatenate([points.T, jnp.zeros((5, n), f32)], axis=0)
    rb = 256 if n % 256 == 0 else n
    ct = 2048 if n % 2048 == 0 else n
    tr = _tile_ranges(b, n, rb, ct)
    nh = n // 2
    b2d = b[None, :]
    bcol = b[:, None]
    idx_lo = _knn(ptsT, b2d, bcol, tr, n, 0, nh)
    idx_hi = _knn(ptsT, b2d, bcol, tr, n, nh // rb, nh)

    x_cat = jnp.concatenate([feature, points], axis=1)
    pts_pad = jnp.concatenate([points, jnp.zeros((n, 5), f32)], axis=1)
    w3T = jnp.concatenate(
        [conv_w[:, cin:].T, jnp.zeros((5, inner), f32)], axis=0)
    g, p2, f3 = _gmat(x_cat, pts_pad, feature, conv_w.T, w3T,
                      lin1_w[:, inner:].T, n, cin, inner)

    ixkm_lo = idx_lo.T.reshape(-1)
    ixkm_hi = idx_hi.T.reshape(-1)
    h_lo = _gather_rows(g, ixkm_lo).reshape(_K, nh, inner)
    h_hi = _gather_rows(g, ixkm_hi).reshape(_K, nh, inner)
    p2_lo, p2_hi = p2[:nh], p2[nh:]
    s = (_stats1(h_lo, p2_lo, nh, inner)
         + _stats1(h_hi, p2_hi, nh, inner))
    nk = jnp.float32(n * _K)
    mean1 = (s[0] - _K * s[3]) / nk
    ex2 = (s[1] - 2.0 * s[2] + _K * s[4]) / nk
    var1 = ex2 - mean1 * mean1
    a1 = bn1_g / jnp.sqrt(var1 + _EPS)
    c1 = bn1_b - a1 * mean1

    y_lo, w_lo = _passb(h_lo, p2_lo, a1[None], c1[None], nh, inner)
    y_hi, w_hi = _passb(h_hi, p2_hi, a1[None], c1[None], nh, inner)
    y = jnp.concatenate([y_lo, y_hi], axis=0)

    h2_lo = _gather_rows(y, ixkm_lo).reshape(_K, nh, inner)
    h2_hi = _gather_rows(y, ixkm_hi).reshape(_K, nh, inner)
    z_lo, acc2_lo = _passc(h2_lo, w_lo.T.reshape(_K, nh, 1), nh, inner)
    z_hi, acc2_hi = _passc(h2_hi, w_hi.T.reshape(_K, nh, 1), nh, inner)
    acc2 = acc2_lo + acc2_hi
    mean2 = acc2[0] / n
    var2 = acc2[1] / n - mean2 * mean2
    a2 = bn2_g / jnp.sqrt(var2 + _EPS)
    c2 = bn2_b - a2 * mean2

    t, acc3 = _passd1(z_lo, z_hi, f3, a2[None], c2[None], lin1_w[:, :inner].T,
                      lin1_b[None], n, cin, inner)
    mean3 = acc3[0] / n
    var3 = acc3[1] / n - mean3 * mean3
    a3 = bn3_g / jnp.sqrt(var3 + _EPS)
    c3 = bn3_b - a3 * mean3

    return _passd2(t, a3[None], c3[None], lin2_w.T, lin2_b[None], n, cin)
